# baseline (speedup 1.0000x reference)
import functools

import jax
import jax.numpy as jnp
from jax import lax
from jax.experimental import pallas as pl
from jax.experimental.pallas import tpu as pltpu
from jax.experimental.pallas import tpu_sc as plsc

N = 10000
NPAD = 10240
E = 320000
NFEAT = 128
NHID = 128
NCLASS = 47
CPAD = 48
BN_EPS = 1e-5

NC = 2
NS = 16
NW = NC * NS
K = 80
EP = 327680
RPT = NPAD // NS
ZB = 16
NB_A = 2
NI = 8
PI = 4
NB_B = 4

CPT = EP // NW // K


def _fill(ref, rows, width, value):
  v = jnp.full((16,), value, ref.dtype)
  for r in range(rows):
    for j in range(width // 16):
      ref[r, pl.ds(j * 16, 16)] = v


def _segsum_feat_kernel():
  mesh = plsc.VectorSubcoreMesh(core_axis_name="c", subcore_axis_name="s")
  out_type = (jax.ShapeDtypeStruct((NC, NPAD, NFEAT), jnp.float32),
              jax.ShapeDtypeStruct((NC, NPAD, 16), jnp.float32))
  scratch = [
      pltpu.VMEM((NI, K), jnp.int32),
      pltpu.VMEM((NI, K), jnp.int32),
      pltpu.VMEM((NB_A, K, NFEAT), jnp.float32),
      pltpu.VMEM((ZB, NFEAT), jnp.float32),
      pltpu.VMEM((K, 16), jnp.float32),
      pltpu.VMEM((ZB, 16), jnp.float32),
      pltpu.VMEM_SHARED((NPAD, NFEAT), jnp.float32),
      pltpu.VMEM_SHARED((NPAD, 16), jnp.float32),
  ] + [pltpu.SemaphoreType.DMA] * (2 * NB_A + 2 * NI + 1)

  def body(feat, src, dst, out, deg_out, sidx, didx, rows, zbuf, ones,
           dzbuf, acc, dacc, *sems):
    gsem = sems[:NB_A]
    ssem = sems[NB_A:2 * NB_A]
    isems = sems[2 * NB_A:2 * NB_A + NI]
    isemd = sems[2 * NB_A + NI:2 * NB_A + 2 * NI]
    dsem = sems[2 * NB_A + 2 * NI]
    c = lax.axis_index("c")
    s = lax.axis_index("s")
    w = c * NS + s
    row0 = w * CPT

    _fill(zbuf, ZB, NFEAT, 0.0)
    _fill(ones, K, 16, 1.0)
    _fill(dzbuf, ZB, 16, 0.0)

    r0 = s * RPT

    def zero_loop(i, _):
      pltpu.sync_copy(zbuf, acc.at[pl.ds(r0 + i * ZB, ZB)])
      pltpu.sync_copy(dzbuf, dacc.at[pl.ds(r0 + i * ZB, ZB)])
      return 0

    lax.fori_loop(0, RPT // ZB, zero_loop, 0)
    plsc.subcore_barrier()

    for j in range(PI):
      pltpu.async_copy(src.at[row0 + j], sidx.at[j], isems[j])
      pltpu.async_copy(dst.at[row0 + j], didx.at[j], isemd[j])
    pltpu.make_async_copy(src.at[row0], sidx.at[0], isems[0]).wait()
    pltpu.make_async_copy(dst.at[row0], didx.at[0], isemd[0]).wait()
    pltpu.async_copy(feat.at[sidx.at[0]], rows.at[0], gsem[0])

    def outer(g, _):
      for j in range(NI):
        cs = g * NI + j
        br = j % NB_A
        bn = (br + 1) % NB_A
        jn = (j + 1) % NI
        jp = (j + PI) % NI

        @pl.when(jnp.logical_and(cs >= 1, cs < CPT - 1))
        def _():
          pltpu.make_async_copy(rows.at[bn], acc.at[didx.at[0]],
                                ssem[bn]).wait()

        @pl.when(cs < CPT - PI)
        def _():
          pltpu.async_copy(src.at[row0 + cs + PI], sidx.at[jp], isems[jp])
          pltpu.async_copy(dst.at[row0 + cs + PI], didx.at[jp], isemd[jp])

        @pl.when(cs < CPT - 1)
        def _():
          pltpu.make_async_copy(src.at[row0], sidx.at[jn], isems[jn]).wait()
          pltpu.make_async_copy(dst.at[row0], didx.at[jn], isemd[jn]).wait()
          pltpu.async_copy(feat.at[sidx.at[jn]], rows.at[bn], gsem[bn])

        pltpu.make_async_copy(feat.at[sidx.at[j]], rows.at[br],
                              gsem[br]).wait()
        pltpu.async_copy(rows.at[br], acc.at[didx.at[j]], ssem[br], add=True)
        pltpu.async_copy(ones, dacc.at[didx.at[j]], dsem, add=True)
      return 0

    lax.fori_loop(0, CPT // NI, outer, 0)

    for b in range(NB_A):
      pltpu.make_async_copy(rows.at[b], acc.at[didx.at[0]], ssem[b]).wait()

    def dloop(i, _):
      pltpu.make_async_copy(ones, dacc.at[didx.at[0]], dsem).wait()
      return 0

    lax.fori_loop(0, CPT, dloop, 0)
    plsc.subcore_barrier()

    pltpu.sync_copy(acc.at[pl.ds(r0, RPT)], out.at[c, pl.ds(r0, RPT)])
    pltpu.sync_copy(dacc.at[pl.ds(r0, RPT)], deg_out.at[c, pl.ds(r0, RPT)])

  return pl.kernel(body, out_type=out_type, mesh=mesh,
                   scratch_types=tuple(scratch),
                   compiler_params=pltpu.CompilerParams(
                       use_tc_tiling_on_sc=False))


def _make_segsum(D, with_degree, nb, pf):
  mesh = plsc.VectorSubcoreMesh(core_axis_name="c", subcore_axis_name="s")
  out_type = [jax.ShapeDtypeStruct((NC, NPAD, D), jnp.float32)]
  scratch = [
      pltpu.VMEM((CPT, K), jnp.int32),
      pltpu.VMEM((CPT, K), jnp.int32),
      pltpu.VMEM((nb, K, D), jnp.float32),
      pltpu.VMEM((ZB, D), jnp.float32),
      pltpu.VMEM_SHARED((NPAD, D), jnp.float32),
  ] + [pltpu.SemaphoreType.DMA] * (2 * nb)
  if with_degree:
    out_type.append(jax.ShapeDtypeStruct((NC, NPAD, 16), jnp.float32))
    scratch += [
        pltpu.VMEM((K, 16), jnp.float32),
        pltpu.VMEM((ZB, 16), jnp.float32),
        pltpu.VMEM_SHARED((NPAD, 16), jnp.float32),
        pltpu.SemaphoreType.DMA,
    ]

  def body(feat, src, dst, *refs):
    if with_degree:
      out, deg_out, sidx, didx, rows, zbuf, acc = refs[:7]
      rest = refs[7:]
      ones, dzbuf, dacc, dsem = rest[2 * nb:]
    else:
      out, sidx, didx, rows, zbuf, acc = refs[:6]
      rest = refs[6:]
    gsem = rest[:nb]
    ssem = rest[nb:2 * nb]
    c = lax.axis_index("c")
    s = lax.axis_index("s")
    w = c * NS + s

    _fill(zbuf, ZB, D, 0.0)
    if with_degree:
      _fill(ones, K, 16, 1.0)
      _fill(dzbuf, ZB, 16, 0.0)

    pltpu.sync_copy(src.at[pl.ds(w * CPT, CPT)], sidx)
    pltpu.sync_copy(dst.at[pl.ds(w * CPT, CPT)], didx)

    r0 = s * RPT

    def zero_loop(i, _):
      pltpu.sync_copy(zbuf, acc.at[pl.ds(r0 + i * ZB, ZB)])
      if with_degree:
        pltpu.sync_copy(dzbuf, dacc.at[pl.ds(r0 + i * ZB, ZB)])
      return 0

    lax.fori_loop(0, RPT // ZB, zero_loop, 0)
    plsc.subcore_barrier()

    for b in range(pf):
      pltpu.async_copy(feat.at[sidx.at[b]], rows.at[b], gsem[b])

    def outer(g, _):
      for b in range(nb):
        cs = g * nb + b
        bg = (b + pf) % nb

        @pl.when(jnp.logical_and(cs >= nb - pf, cs < CPT - pf))
        def _():
          pltpu.make_async_copy(rows.at[bg], acc.at[didx.at[0]],
                                ssem[bg]).wait()

        @pl.when(cs < CPT - pf)
        def _():
          pltpu.async_copy(feat.at[sidx.at[cs + pf]], rows.at[bg], gsem[bg])

        pltpu.make_async_copy(feat.at[sidx.at[cs]], rows.at[b],
                              gsem[b]).wait()
        pltpu.async_copy(rows.at[b], acc.at[didx.at[cs]], ssem[b], add=True)
        if with_degree:
          pltpu.async_copy(ones, dacc.at[didx.at[cs]], dsem, add=True)
      return 0

    lax.fori_loop(0, CPT // nb, outer, 0)

    for b in range(nb):
      pltpu.make_async_copy(rows.at[b], acc.at[didx.at[0]], ssem[b]).wait()
    if with_degree:
      def dloop(i, _):
        pltpu.make_async_copy(ones, dacc.at[didx.at[0]], dsem).wait()
        return 0
      lax.fori_loop(0, CPT, dloop, 0)
    plsc.subcore_barrier()

    pltpu.sync_copy(acc.at[pl.ds(r0, RPT)], out.at[c, pl.ds(r0, RPT)])
    if with_degree:
      pltpu.sync_copy(dacc.at[pl.ds(r0, RPT)], deg_out.at[c, pl.ds(r0, RPT)])

  return pl.kernel(body, out_type=tuple(out_type), mesh=mesh,
                   scratch_types=tuple(scratch),
                   compiler_params=pltpu.CompilerParams(
                       use_tc_tiling_on_sc=False))


_segsum_feat = _segsum_feat_kernel()
_segsum_cls = _make_segsum(CPAD, False, NB_B, 2)

BR = 1024


def _dense0_body(s0p, degp, x, wl0, bl0, wr0, scale, shift, wl1, h_out, q_out):
  deg = degp[0, :, 0:1] + degp[1, :, 0:1]
  mean = (s0p[0] + s0p[1]) / jnp.maximum(deg, 1.0)
  z = (jax.lax.dot(mean, wl0[...], preferred_element_type=jnp.float32)
       + bl0[...]
       + jax.lax.dot(x[...], wr0[...], preferred_element_type=jnp.float32))
  h = jnp.maximum(z * scale[...] + shift[...], 0.0)
  h_out[...] = h
  q_out[...] = jax.lax.dot(h, wl1[...], preferred_element_type=jnp.float32)


def _dense0(s0p, degp, x, wl0, bl0, wr0, scale, shift, wl1):
  grid = (NPAD // BR,)
  return pl.pallas_call(
      _dense0_body,
      grid=grid,
      in_specs=[
          pl.BlockSpec((NC, BR, NFEAT), lambda i: (0, i, 0)),
          pl.BlockSpec((NC, BR, 16), lambda i: (0, i, 0)),
          pl.BlockSpec((BR, NFEAT), lambda i: (i, 0)),
          pl.BlockSpec((NFEAT, NHID), lambda i: (0, 0)),
          pl.BlockSpec((1, NHID), lambda i: (0, 0)),
          pl.BlockSpec((NFEAT, NHID), lambda i: (0, 0)),
          pl.BlockSpec((1, NHID), lambda i: (0, 0)),
          pl.BlockSpec((1, NHID), lambda i: (0, 0)),
          pl.BlockSpec((NHID, CPAD), lambda i: (0, 0)),
      ],
      out_specs=[
          pl.BlockSpec((BR, NHID), lambda i: (i, 0)),
          pl.BlockSpec((BR, CPAD), lambda i: (i, 0)),
      ],
      out_shape=[
          jax.ShapeDtypeStruct((NPAD, NHID), jnp.float32),
          jax.ShapeDtypeStruct((NPAD, CPAD), jnp.float32),
      ],
  )(s0p, degp, x, wl0, bl0, wr0, scale, shift, wl1)


def _dense1_body(s1p, degp, h, wr1, bl1, out):
  ssum = s1p[0] + s1p[1]
  deg = degp[0, :, 0:1] + degp[1, :, 0:1]
  z = (ssum / jnp.maximum(deg, 1.0) + bl1[...]
       + jax.lax.dot(h[...], wr1[...], preferred_element_type=jnp.float32))
  mask = lax.broadcasted_iota(jnp.int32, (1, CPAD), 1) < NCLASS
  z = jnp.where(mask, z, -1e30)
  m = jnp.max(z, axis=1, keepdims=True)
  ez = jnp.exp(z - m)
  lse = jnp.log(jnp.sum(ez, axis=1, keepdims=True))
  out[...] = z - m - lse


def _dense1(s1p, degp, h, wr1, bl1):
  grid = (NPAD // BR,)
  return pl.pallas_call(
      _dense1_body,
      grid=grid,
      in_specs=[
          pl.BlockSpec((NC, BR, CPAD), lambda i: (0, i, 0)),
          pl.BlockSpec((NC, BR, 16), lambda i: (0, i, 0)),
          pl.BlockSpec((BR, NHID), lambda i: (i, 0)),
          pl.BlockSpec((NHID, CPAD), lambda i: (0, 0)),
          pl.BlockSpec((1, CPAD), lambda i: (0, 0)),
      ],
      out_specs=pl.BlockSpec((BR, CPAD), lambda i: (i, 0)),
      out_shape=jax.ShapeDtypeStruct((NPAD, CPAD), jnp.float32),
  )(s1p, degp, h, wr1, bl1)


def kernel(x, edge_index, W_l0, b_l0, W_r0, gamma0, beta0, W_l1, b_l1, W_r1):
  src = jnp.concatenate(
      [edge_index[0], jnp.zeros((EP - E,), jnp.int32)]).reshape(EP // K, K)
  dst = jnp.concatenate(
      [edge_index[1], jnp.full((EP - E,), N, jnp.int32)]).reshape(EP // K, K)
  s0p, degp = _segsum_feat(x, src, dst)

  scale = (gamma0 / jnp.sqrt(1.0 + BN_EPS)).reshape(1, NHID)
  shift = beta0.reshape(1, NHID)
  wl1 = jnp.pad(W_l1, ((0, 0), (0, CPAD - NCLASS)))
  xpad = jnp.pad(x, ((0, NPAD - N), (0, 0)))
  h, q = _dense0(s0p, degp, xpad, W_l0, b_l0.reshape(1, NHID), W_r0,
                 scale, shift, wl1)

  (s1p,) = _segsum_cls(q, src, dst)

  wr1 = jnp.pad(W_r1, ((0, 0), (0, CPAD - NCLASS)))
  bl1 = jnp.pad(b_l1, (0, CPAD - NCLASS)).reshape(1, CPAD)
  out = _dense1(s1p, degp, h, wr1, bl1)
  return out[:N, :NCLASS]


# speedup vs baseline: 2.9653x; 2.9653x over previous
import functools

import jax
import jax.numpy as jnp
from jax import lax
from jax.experimental import pallas as pl
from jax.experimental.pallas import tpu as pltpu
from jax.experimental.pallas import tpu_sc as plsc

N = 10000
NPAD = 10240
E = 320000
NFEAT = 128
NHID = 128
NCLASS = 47
CPAD = 48
BN_EPS = 1e-5

NC = 2
NS = 16
NW = NC * NS
K = 80
RPT = NPAD // NS
ZB = 40
NB_A = 5
NB_B = 5
PF_A = 2
PF_B = 2

CPT_A = E // NS // K
CPT_B = E // NW // K


def _fill(ref, rows, width, value):
  v = jnp.full((16,), value, ref.dtype)
  for r in range(rows):
    for j in range(width // 16):
      ref[r, pl.ds(j * 16, 16)] = v


def _segsum_feat_kernel():
  mesh = plsc.VectorSubcoreMesh(core_axis_name="c", subcore_axis_name="s")
  out_type = (jax.ShapeDtypeStruct((NPAD, NFEAT), jnp.float32),
              jax.ShapeDtypeStruct((NPAD, 16), jnp.float32))
  scratch = [
      pltpu.VMEM((CPT_A, K), jnp.int32),
      pltpu.VMEM((CPT_A, K), jnp.int32),
      pltpu.VMEM((NB_A, K, 64), jnp.float32),
      pltpu.VMEM((ZB, 64), jnp.float32),
      pltpu.VMEM((K, 16), jnp.float32),
      pltpu.VMEM((ZB, 16), jnp.float32),
      pltpu.VMEM_SHARED((NPAD, 64), jnp.float32),
      pltpu.VMEM_SHARED((NPAD, 16), jnp.float32),
  ] + [pltpu.SemaphoreType.DMA] * (2 * NB_A + 1)

  def body(feat2, src, dst, out, deg_out, sidx, didx, rows, zbuf, ones,
           dzbuf, acc, dacc, *sems):
    gsem = sems[:NB_A]
    ssem = sems[NB_A:2 * NB_A]
    dsem = sems[2 * NB_A]
    c = lax.axis_index("c")
    s = lax.axis_index("s")
    on_c0 = c == 0

    _fill(zbuf, ZB, 64, 0.0)
    _fill(ones, K, 16, 1.0)
    _fill(dzbuf, ZB, 16, 0.0)

    pltpu.sync_copy(src.at[pl.ds(s * CPT_A, CPT_A)], sidx)
    pltpu.sync_copy(dst.at[pl.ds(s * CPT_A, CPT_A)], didx)

    def xform(r, _):
      for j in range(K // 16):
        sl = pl.ds(j * 16, 16)
        sidx[r, sl] = sidx[r, sl] * 2 + c
      return 0

    lax.fori_loop(0, CPT_A, xform, 0)

    r0 = s * RPT

    def zero_loop(i, _):
      pltpu.sync_copy(zbuf, acc.at[pl.ds(r0 + i * ZB, ZB)])

      @pl.when(on_c0)
      def _():
        pltpu.sync_copy(dzbuf, dacc.at[pl.ds(r0 + i * ZB, ZB)])
      return 0

    lax.fori_loop(0, RPT // ZB, zero_loop, 0)
    plsc.subcore_barrier()

    for b in range(PF_A):
      pltpu.async_copy(feat2.at[sidx.at[b]], rows.at[b], gsem[b])

    def outer(g, _):
      for b in range(NB_A):
        cs = g * NB_A + b
        bg = (b + PF_A) % NB_A

        @pl.when(jnp.logical_and(cs >= NB_A - PF_A, cs < CPT_A - PF_A))
        def _():
          pltpu.make_async_copy(rows.at[bg], acc.at[didx.at[0]],
                                ssem[bg]).wait()

        @pl.when(cs < CPT_A - PF_A)
        def _():
          pltpu.async_copy(feat2.at[sidx.at[cs + PF_A]], rows.at[bg],
                           gsem[bg])

        pltpu.make_async_copy(feat2.at[sidx.at[cs]], rows.at[b],
                              gsem[b]).wait()
        pltpu.async_copy(rows.at[b], acc.at[didx.at[cs]], ssem[b], add=True)

        @pl.when(on_c0)
        def _():
          pltpu.async_copy(ones, dacc.at[didx.at[cs]], dsem, add=True)
      return 0

    lax.fori_loop(0, CPT_A // NB_A, outer, 0)

    for b in range(NB_A):
      pltpu.make_async_copy(rows.at[b], acc.at[didx.at[0]], ssem[b]).wait()

    @pl.when(on_c0)
    def _():
      def dloop(i, _):
        pltpu.make_async_copy(ones, dacc.at[didx.at[0]], dsem).wait()
        return 0
      lax.fori_loop(0, CPT_A, dloop, 0)

    plsc.subcore_barrier()

    pltpu.sync_copy(acc.at[pl.ds(r0, RPT)],
                    out.at[pl.ds(r0, RPT), pl.ds(c * 64, 64)])

    @pl.when(on_c0)
    def _():
      pltpu.sync_copy(dacc.at[pl.ds(r0, RPT)], deg_out.at[pl.ds(r0, RPT)])

  return pl.kernel(body, out_type=out_type, mesh=mesh,
                   scratch_types=tuple(scratch),
                   compiler_params=pltpu.CompilerParams(
                       use_tc_tiling_on_sc=False))


def _segsum_cls_kernel():
  mesh = plsc.VectorSubcoreMesh(core_axis_name="c", subcore_axis_name="s")
  out_type = jax.ShapeDtypeStruct((NC, NPAD, CPAD), jnp.float32)
  scratch = [
      pltpu.VMEM((CPT_B, K), jnp.int32),
      pltpu.VMEM((CPT_B, K), jnp.int32),
      pltpu.VMEM((NB_B, K, CPAD), jnp.float32),
      pltpu.VMEM((ZB, CPAD), jnp.float32),
      pltpu.VMEM_SHARED((NPAD, CPAD), jnp.float32),
  ] + [pltpu.SemaphoreType.DMA] * (2 * NB_B)

  def body(feat, src, dst, out, sidx, didx, rows, zbuf, acc, *sems):
    gsem = sems[:NB_B]
    ssem = sems[NB_B:2 * NB_B]
    c = lax.axis_index("c")
    s = lax.axis_index("s")
    w = c * NS + s

    _fill(zbuf, ZB, CPAD, 0.0)

    pltpu.sync_copy(src.at[pl.ds(w * CPT_B, CPT_B)], sidx)
    pltpu.sync_copy(dst.at[pl.ds(w * CPT_B, CPT_B)], didx)

    r0 = s * RPT

    def zero_loop(i, _):
      pltpu.sync_copy(zbuf, acc.at[pl.ds(r0 + i * ZB, ZB)])
      return 0

    lax.fori_loop(0, RPT // ZB, zero_loop, 0)
    plsc.subcore_barrier()

    for b in range(PF_B):
      pltpu.async_copy(feat.at[sidx.at[b]], rows.at[b], gsem[b])

    def outer(g, _):
      for b in range(NB_B):
        cs = g * NB_B + b
        bg = (b + PF_B) % NB_B

        @pl.when(jnp.logical_and(cs >= NB_B - PF_B, cs < CPT_B - PF_B))
        def _():
          pltpu.make_async_copy(rows.at[bg], acc.at[didx.at[0]],
                                ssem[bg]).wait()

        @pl.when(cs < CPT_B - PF_B)
        def _():
          pltpu.async_copy(feat.at[sidx.at[cs + PF_B]], rows.at[bg],
                           gsem[bg])

        pltpu.make_async_copy(feat.at[sidx.at[cs]], rows.at[b],
                              gsem[b]).wait()
        pltpu.async_copy(rows.at[b], acc.at[didx.at[cs]], ssem[b], add=True)
      return 0

    lax.fori_loop(0, CPT_B // NB_B, outer, 0)

    for b in range(NB_B):
      pltpu.make_async_copy(rows.at[b], acc.at[didx.at[0]], ssem[b]).wait()
    plsc.subcore_barrier()

    pltpu.sync_copy(acc.at[pl.ds(r0, RPT)], out.at[c, pl.ds(r0, RPT)])

  return pl.kernel(body, out_type=out_type, mesh=mesh,
                   scratch_types=tuple(scratch),
                   compiler_params=pltpu.CompilerParams(
                       use_tc_tiling_on_sc=False))


_segsum_feat = _segsum_feat_kernel()
_segsum_cls = _segsum_cls_kernel()

BR = 1024


def _dense0_body(s0, degp, x, wl0, bl0, wr0, scale, shift, wl1, h_out, q_out):
  deg = degp[:, 0:1]
  mean = s0[...] / jnp.maximum(deg, 1.0)
  z = (jax.lax.dot(mean, wl0[...], preferred_element_type=jnp.float32)
       + bl0[...]
       + jax.lax.dot(x[...], wr0[...], preferred_element_type=jnp.float32))
  h = jnp.maximum(z * scale[...] + shift[...], 0.0)
  h_out[...] = h
  q_out[...] = jax.lax.dot(h, wl1[...], preferred_element_type=jnp.float32)


def _dense0(s0, degp, x, wl0, bl0, wr0, scale, shift, wl1):
  grid = (NPAD // BR,)
  return pl.pallas_call(
      _dense0_body,
      grid=grid,
      in_specs=[
          pl.BlockSpec((BR, NFEAT), lambda i: (i, 0)),
          pl.BlockSpec((BR, 16), lambda i: (i, 0)),
          pl.BlockSpec((BR, NFEAT), lambda i: (i, 0)),
          pl.BlockSpec((NFEAT, NHID), lambda i: (0, 0)),
          pl.BlockSpec((1, NHID), lambda i: (0, 0)),
          pl.BlockSpec((NFEAT, NHID), lambda i: (0, 0)),
          pl.BlockSpec((1, NHID), lambda i: (0, 0)),
          pl.BlockSpec((1, NHID), lambda i: (0, 0)),
          pl.BlockSpec((NHID, CPAD), lambda i: (0, 0)),
      ],
      out_specs=[
          pl.BlockSpec((BR, NHID), lambda i: (i, 0)),
          pl.BlockSpec((BR, CPAD), lambda i: (i, 0)),
      ],
      out_shape=[
          jax.ShapeDtypeStruct((NPAD, NHID), jnp.float32),
          jax.ShapeDtypeStruct((NPAD, CPAD), jnp.float32),
      ],
  )(s0, degp, x, wl0, bl0, wr0, scale, shift, wl1)


def _dense1_body(s1p, degp, h, wr1, bl1, out):
  ssum = s1p[0] + s1p[1]
  deg = degp[:, 0:1]
  z = (ssum / jnp.maximum(deg, 1.0) + bl1[...]
       + jax.lax.dot(h[...], wr1[...], preferred_element_type=jnp.float32))
  mask = lax.broadcasted_iota(jnp.int32, (1, CPAD), 1) < NCLASS
  z = jnp.where(mask, z, -1e30)
  m = jnp.max(z, axis=1, keepdims=True)
  ez = jnp.exp(z - m)
  lse = jnp.log(jnp.sum(ez, axis=1, keepdims=True))
  out[...] = z - m - lse


def _dense1(s1p, degp, h, wr1, bl1):
  grid = (NPAD // BR,)
  return pl.pallas_call(
      _dense1_body,
      grid=grid,
      in_specs=[
          pl.BlockSpec((NC, BR, CPAD), lambda i: (0, i, 0)),
          pl.BlockSpec((BR, 16), lambda i: (i, 0)),
          pl.BlockSpec((BR, NHID), lambda i: (i, 0)),
          pl.BlockSpec((NHID, CPAD), lambda i: (0, 0)),
          pl.BlockSpec((1, CPAD), lambda i: (0, 0)),
      ],
      out_specs=pl.BlockSpec((BR, CPAD), lambda i: (i, 0)),
      out_shape=jax.ShapeDtypeStruct((NPAD, CPAD), jnp.float32),
  )(s1p, degp, h, wr1, bl1)


def kernel(x, edge_index, W_l0, b_l0, W_r0, gamma0, beta0, W_l1, b_l1, W_r1):
  src = edge_index[0].reshape(E // K, K)
  dst = edge_index[1].reshape(E // K, K)
  x2 = x.reshape(2 * N, 64)
  s0, degp = _segsum_feat(x2, src, dst)

  scale = (gamma0 / jnp.sqrt(1.0 + BN_EPS)).reshape(1, NHID)
  shift = beta0.reshape(1, NHID)
  wl1 = jnp.pad(W_l1, ((0, 0), (0, CPAD - NCLASS)))
  xpad = jnp.pad(x, ((0, NPAD - N), (0, 0)))
  h, q = _dense0(s0, degp, xpad, W_l0, b_l0.reshape(1, NHID), W_r0,
                 scale, shift, wl1)

  s1p = _segsum_cls(q, src, dst)

  wr1 = jnp.pad(W_r1, ((0, 0), (0, CPAD - NCLASS)))
  bl1 = jnp.pad(b_l1, (0, CPAD - NCLASS)).reshape(1, CPAD)
  out = _dense1(s1p, degp, h, wr1, bl1)
  return out[:N, :NCLASS]


# speedup vs baseline: 3.0912x; 1.0425x over previous
import functools

import jax
import jax.numpy as jnp
from jax import lax
from jax.experimental import pallas as pl
from jax.experimental.pallas import tpu as pltpu
from jax.experimental.pallas import tpu_sc as plsc

N = 10000
NPAD = 10240
E = 320000
NFEAT = 128
NHID = 128
NCLASS = 47
CPAD = 48
BN_EPS = 1e-5

NC = 2
NS = 16
NW = NC * NS
K = 80
RPT = NPAD // NS
ZB = 40
NB_A = 5
NB_B = 5
PF_A = 3
PF_B = 3

CPT_A = E // NS // K
CPT_B = E // NW // K


def _fill(ref, rows, width, value):
  v = jnp.full((16,), value, ref.dtype)
  for r in range(rows):
    for j in range(width // 16):
      ref[r, pl.ds(j * 16, 16)] = v


def _segsum_feat_kernel():
  mesh = plsc.VectorSubcoreMesh(core_axis_name="c", subcore_axis_name="s")
  out_type = (jax.ShapeDtypeStruct((NPAD, NFEAT), jnp.float32),
              jax.ShapeDtypeStruct((NPAD, 16), jnp.float32))
  scratch = [
      pltpu.VMEM((CPT_A, K), jnp.int32),
      pltpu.VMEM((CPT_A, K), jnp.int32),
      pltpu.VMEM((NB_A, K, 64), jnp.float32),
      pltpu.VMEM((ZB, 64), jnp.float32),
      pltpu.VMEM((K, 16), jnp.float32),
      pltpu.VMEM((ZB, 16), jnp.float32),
      pltpu.VMEM_SHARED((NPAD, 64), jnp.float32),
      pltpu.VMEM_SHARED((NPAD, 16), jnp.float32),
  ] + [pltpu.SemaphoreType.DMA] * (2 * NB_A + 1)

  def body(feat2, src, dst, out, deg_out, sidx, didx, rows, zbuf, ones,
           dzbuf, acc, dacc, *sems):
    gsem = sems[:NB_A]
    ssem = sems[NB_A:2 * NB_A]
    dsem = sems[2 * NB_A]
    c = lax.axis_index("c")
    s = lax.axis_index("s")
    on_c0 = c == 0

    _fill(zbuf, ZB, 64, 0.0)
    _fill(ones, K, 16, 1.0)
    _fill(dzbuf, ZB, 16, 0.0)

    pltpu.sync_copy(src.at[pl.ds(s * CPT_A, CPT_A)], sidx)
    pltpu.sync_copy(dst.at[pl.ds(s * CPT_A, CPT_A)], didx)

    def xform(r, _):
      for j in range(K // 16):
        sl = pl.ds(j * 16, 16)
        sidx[r, sl] = sidx[r, sl] * 2 + c
      return 0

    lax.fori_loop(0, CPT_A, xform, 0)

    r0 = s * RPT

    def zero_loop(i, _):
      pltpu.sync_copy(zbuf, acc.at[pl.ds(r0 + i * ZB, ZB)])

      @pl.when(on_c0)
      def _():
        pltpu.sync_copy(dzbuf, dacc.at[pl.ds(r0 + i * ZB, ZB)])
      return 0

    lax.fori_loop(0, RPT // ZB, zero_loop, 0)
    plsc.subcore_barrier()

    for b in range(PF_A):
      pltpu.async_copy(feat2.at[sidx.at[b]], rows.at[b], gsem[b])

    def outer(g, _):
      for b in range(NB_A):
        cs = g * NB_A + b
        bg = (b + PF_A) % NB_A

        @pl.when(jnp.logical_and(cs >= NB_A - PF_A, cs < CPT_A - PF_A))
        def _():
          pltpu.make_async_copy(rows.at[bg], acc.at[didx.at[0]],
                                ssem[bg]).wait()

        @pl.when(cs < CPT_A - PF_A)
        def _():
          pltpu.async_copy(feat2.at[sidx.at[cs + PF_A]], rows.at[bg],
                           gsem[bg])

        pltpu.make_async_copy(feat2.at[sidx.at[cs]], rows.at[b],
                              gsem[b]).wait()
        pltpu.async_copy(rows.at[b], acc.at[didx.at[cs]], ssem[b], add=True)

        @pl.when(on_c0)
        def _():
          pltpu.async_copy(ones, dacc.at[didx.at[cs]], dsem, add=True)
      return 0

    lax.fori_loop(0, CPT_A // NB_A, outer, 0)

    for b in range(NB_A):
      pltpu.make_async_copy(rows.at[b], acc.at[didx.at[0]], ssem[b]).wait()

    @pl.when(on_c0)
    def _():
      def dloop(i, _):
        pltpu.make_async_copy(ones, dacc.at[didx.at[0]], dsem).wait()
        return 0
      lax.fori_loop(0, CPT_A, dloop, 0)

    plsc.subcore_barrier()

    pltpu.sync_copy(acc.at[pl.ds(r0, RPT)],
                    out.at[pl.ds(r0, RPT), pl.ds(c * 64, 64)])

    @pl.when(on_c0)
    def _():
      pltpu.sync_copy(dacc.at[pl.ds(r0, RPT)], deg_out.at[pl.ds(r0, RPT)])

  return pl.kernel(body, out_type=out_type, mesh=mesh,
                   scratch_types=tuple(scratch),
                   compiler_params=pltpu.CompilerParams(
                       use_tc_tiling_on_sc=False))


def _segsum_cls_kernel():
  mesh = plsc.VectorSubcoreMesh(core_axis_name="c", subcore_axis_name="s")
  out_type = jax.ShapeDtypeStruct((NC, NPAD, CPAD), jnp.float32)
  scratch = [
      pltpu.VMEM((CPT_B, K), jnp.int32),
      pltpu.VMEM((CPT_B, K), jnp.int32),
      pltpu.VMEM((NB_B, K, CPAD), jnp.float32),
      pltpu.VMEM((ZB, CPAD), jnp.float32),
      pltpu.VMEM_SHARED((NPAD, CPAD), jnp.float32),
  ] + [pltpu.SemaphoreType.DMA] * (2 * NB_B)

  def body(feat, src, dst, out, sidx, didx, rows, zbuf, acc, *sems):
    gsem = sems[:NB_B]
    ssem = sems[NB_B:2 * NB_B]
    c = lax.axis_index("c")
    s = lax.axis_index("s")
    w = c * NS + s

    _fill(zbuf, ZB, CPAD, 0.0)

    pltpu.sync_copy(src.at[pl.ds(w * CPT_B, CPT_B)], sidx)
    pltpu.sync_copy(dst.at[pl.ds(w * CPT_B, CPT_B)], didx)

    r0 = s * RPT

    def zero_loop(i, _):
      pltpu.sync_copy(zbuf, acc.at[pl.ds(r0 + i * ZB, ZB)])
      return 0

    lax.fori_loop(0, RPT // ZB, zero_loop, 0)
    plsc.subcore_barrier()

    for b in range(PF_B):
      pltpu.async_copy(feat.at[sidx.at[b]], rows.at[b], gsem[b])

    def outer(g, _):
      for b in range(NB_B):
        cs = g * NB_B + b
        bg = (b + PF_B) % NB_B

        @pl.when(jnp.logical_and(cs >= NB_B - PF_B, cs < CPT_B - PF_B))
        def _():
          pltpu.make_async_copy(rows.at[bg], acc.at[didx.at[0]],
                                ssem[bg]).wait()

        @pl.when(cs < CPT_B - PF_B)
        def _():
          pltpu.async_copy(feat.at[sidx.at[cs + PF_B]], rows.at[bg],
                           gsem[bg])

        pltpu.make_async_copy(feat.at[sidx.at[cs]], rows.at[b],
                              gsem[b]).wait()
        pltpu.async_copy(rows.at[b], acc.at[didx.at[cs]], ssem[b], add=True)
      return 0

    lax.fori_loop(0, CPT_B // NB_B, outer, 0)

    for b in range(NB_B):
      pltpu.make_async_copy(rows.at[b], acc.at[didx.at[0]], ssem[b]).wait()
    plsc.subcore_barrier()

    pltpu.sync_copy(acc.at[pl.ds(r0, RPT)], out.at[c, pl.ds(r0, RPT)])

  return pl.kernel(body, out_type=out_type, mesh=mesh,
                   scratch_types=tuple(scratch),
                   compiler_params=pltpu.CompilerParams(
                       use_tc_tiling_on_sc=False))


_segsum_feat = _segsum_feat_kernel()
_segsum_cls = _segsum_cls_kernel()

BR = 1024


def _dense0_body(s0, degp, x, wl0, bl0, wr0, scale, shift, wl1, h_out, q_out):
  deg = degp[:, 0:1]
  mean = s0[...] / jnp.maximum(deg, 1.0)
  z = (jax.lax.dot(mean, wl0[...], preferred_element_type=jnp.float32)
       + bl0[...]
       + jax.lax.dot(x[...], wr0[...], preferred_element_type=jnp.float32))
  h = jnp.maximum(z * scale[...] + shift[...], 0.0)
  h_out[...] = h
  q_out[...] = jax.lax.dot(h, wl1[...], preferred_element_type=jnp.float32)


def _dense0(s0, degp, x, wl0, bl0, wr0, scale, shift, wl1):
  grid = (NPAD // BR,)
  return pl.pallas_call(
      _dense0_body,
      grid=grid,
      in_specs=[
          pl.BlockSpec((BR, NFEAT), lambda i: (i, 0)),
          pl.BlockSpec((BR, 16), lambda i: (i, 0)),
          pl.BlockSpec((BR, NFEAT), lambda i: (i, 0)),
          pl.BlockSpec((NFEAT, NHID), lambda i: (0, 0)),
          pl.BlockSpec((1, NHID), lambda i: (0, 0)),
          pl.BlockSpec((NFEAT, NHID), lambda i: (0, 0)),
          pl.BlockSpec((1, NHID), lambda i: (0, 0)),
          pl.BlockSpec((1, NHID), lambda i: (0, 0)),
          pl.BlockSpec((NHID, CPAD), lambda i: (0, 0)),
      ],
      out_specs=[
          pl.BlockSpec((BR, NHID), lambda i: (i, 0)),
          pl.BlockSpec((BR, CPAD), lambda i: (i, 0)),
      ],
      out_shape=[
          jax.ShapeDtypeStruct((NPAD, NHID), jnp.float32),
          jax.ShapeDtypeStruct((NPAD, CPAD), jnp.float32),
      ],
  )(s0, degp, x, wl0, bl0, wr0, scale, shift, wl1)


def _dense1_body(s1p, degp, h, wr1, bl1, out):
  ssum = s1p[0] + s1p[1]
  deg = degp[:, 0:1]
  z = (ssum / jnp.maximum(deg, 1.0) + bl1[...]
       + jax.lax.dot(h[...], wr1[...], preferred_element_type=jnp.float32))
  mask = lax.broadcasted_iota(jnp.int32, (1, CPAD), 1) < NCLASS
  z = jnp.where(mask, z, -1e30)
  m = jnp.max(z, axis=1, keepdims=True)
  ez = jnp.exp(z - m)
  lse = jnp.log(jnp.sum(ez, axis=1, keepdims=True))
  out[...] = z - m - lse


def _dense1(s1p, degp, h, wr1, bl1):
  grid = (NPAD // BR,)
  return pl.pallas_call(
      _dense1_body,
      grid=grid,
      in_specs=[
          pl.BlockSpec((NC, BR, CPAD), lambda i: (0, i, 0)),
          pl.BlockSpec((BR, 16), lambda i: (i, 0)),
          pl.BlockSpec((BR, NHID), lambda i: (i, 0)),
          pl.BlockSpec((NHID, CPAD), lambda i: (0, 0)),
          pl.BlockSpec((1, CPAD), lambda i: (0, 0)),
      ],
      out_specs=pl.BlockSpec((BR, CPAD), lambda i: (i, 0)),
      out_shape=jax.ShapeDtypeStruct((NPAD, CPAD), jnp.float32),
  )(s1p, degp, h, wr1, bl1)


def kernel(x, edge_index, W_l0, b_l0, W_r0, gamma0, beta0, W_l1, b_l1, W_r1):
  src = edge_index[0].reshape(E // K, K)
  dst = edge_index[1].reshape(E // K, K)
  x2 = x.reshape(2 * N, 64)
  s0, degp = _segsum_feat(x2, src, dst)

  scale = (gamma0 / jnp.sqrt(1.0 + BN_EPS)).reshape(1, NHID)
  shift = beta0.reshape(1, NHID)
  wl1 = jnp.pad(W_l1, ((0, 0), (0, CPAD - NCLASS)))
  xpad = jnp.pad(x, ((0, NPAD - N), (0, 0)))
  h, q = _dense0(s0, degp, xpad, W_l0, b_l0.reshape(1, NHID), W_r0,
                 scale, shift, wl1)

  s1p = _segsum_cls(q, src, dst)

  wr1 = jnp.pad(W_r1, ((0, 0), (0, CPAD - NCLASS)))
  bl1 = jnp.pad(b_l1, (0, CPAD - NCLASS)).reshape(1, CPAD)
  out = _dense1(s1p, degp, h, wr1, bl1)
  return out[:N, :NCLASS]


# speedup vs baseline: 3.1258x; 1.0112x over previous
import functools

import jax
import jax.numpy as jnp
from jax import lax
from jax.experimental import pallas as pl
from jax.experimental.pallas import tpu as pltpu
from jax.experimental.pallas import tpu_sc as plsc

N = 10000
NPAD = 10240
E = 320000
NFEAT = 128
NHID = 128
NCLASS = 47
CPAD = 48
BN_EPS = 1e-5

NC = 2
NS = 16
NW = NC * NS
K = 80
RPT = NPAD // NS
ZB = 40
NB_A = 5
NB_B = 5
PF_A = 4
PF_B = 4

CPT_A = E // NS // K
CPT_B = E // NW // K


def _fill(ref, rows, width, value):
  v = jnp.full((16,), value, ref.dtype)
  for r in range(rows):
    for j in range(width // 16):
      ref[r, pl.ds(j * 16, 16)] = v


def _segsum_feat_kernel():
  mesh = plsc.VectorSubcoreMesh(core_axis_name="c", subcore_axis_name="s")
  out_type = (jax.ShapeDtypeStruct((NPAD, NFEAT), jnp.float32),
              jax.ShapeDtypeStruct((NPAD, 16), jnp.float32))
  scratch = [
      pltpu.VMEM((CPT_A, K), jnp.int32),
      pltpu.VMEM((CPT_A, K), jnp.int32),
      pltpu.VMEM((NB_A, K, 64), jnp.float32),
      pltpu.VMEM((ZB, 64), jnp.float32),
      pltpu.VMEM((K, 16), jnp.float32),
      pltpu.VMEM((ZB, 16), jnp.float32),
      pltpu.VMEM_SHARED((NPAD, 64), jnp.float32),
      pltpu.VMEM_SHARED((NPAD, 16), jnp.float32),
  ] + [pltpu.SemaphoreType.DMA] * (2 * NB_A + 1)

  def body(feat2, src, dst, out, deg_out, sidx, didx, rows, zbuf, ones,
           dzbuf, acc, dacc, *sems):
    gsem = sems[:NB_A]
    ssem = sems[NB_A:2 * NB_A]
    dsem = sems[2 * NB_A]
    c = lax.axis_index("c")
    s = lax.axis_index("s")
    on_c0 = c == 0

    _fill(zbuf, ZB, 64, 0.0)
    _fill(ones, K, 16, 1.0)
    _fill(dzbuf, ZB, 16, 0.0)

    pltpu.sync_copy(src.at[pl.ds(s * CPT_A, CPT_A)], sidx)
    pltpu.sync_copy(dst.at[pl.ds(s * CPT_A, CPT_A)], didx)

    def xform(r, _):
      for j in range(K // 16):
        sl = pl.ds(j * 16, 16)
        sidx[r, sl] = sidx[r, sl] * 2 + c
      return 0

    lax.fori_loop(0, CPT_A, xform, 0)

    r0 = s * RPT

    def zero_loop(i, _):
      pltpu.sync_copy(zbuf, acc.at[pl.ds(r0 + i * ZB, ZB)])

      @pl.when(on_c0)
      def _():
        pltpu.sync_copy(dzbuf, dacc.at[pl.ds(r0 + i * ZB, ZB)])
      return 0

    lax.fori_loop(0, RPT // ZB, zero_loop, 0)
    plsc.subcore_barrier()

    for b in range(PF_A):
      pltpu.async_copy(feat2.at[sidx.at[b]], rows.at[b], gsem[b])

    def outer(g, _):
      for b in range(NB_A):
        cs = g * NB_A + b
        bg = (b + PF_A) % NB_A

        @pl.when(jnp.logical_and(cs >= NB_A - PF_A, cs < CPT_A - PF_A))
        def _():
          pltpu.make_async_copy(rows.at[bg], acc.at[didx.at[0]],
                                ssem[bg]).wait()

        @pl.when(cs < CPT_A - PF_A)
        def _():
          pltpu.async_copy(feat2.at[sidx.at[cs + PF_A]], rows.at[bg],
                           gsem[bg])

        pltpu.make_async_copy(feat2.at[sidx.at[cs]], rows.at[b],
                              gsem[b]).wait()
        pltpu.async_copy(rows.at[b], acc.at[didx.at[cs]], ssem[b], add=True)

        @pl.when(on_c0)
        def _():
          pltpu.async_copy(ones, dacc.at[didx.at[cs]], dsem, add=True)
      return 0

    lax.fori_loop(0, CPT_A // NB_A, outer, 0)

    for b in range(NB_A):
      pltpu.make_async_copy(rows.at[b], acc.at[didx.at[0]], ssem[b]).wait()

    @pl.when(on_c0)
    def _():
      def dloop(i, _):
        pltpu.make_async_copy(ones, dacc.at[didx.at[0]], dsem).wait()
        return 0
      lax.fori_loop(0, CPT_A, dloop, 0)

    plsc.subcore_barrier()

    pltpu.sync_copy(acc.at[pl.ds(r0, RPT)],
                    out.at[pl.ds(r0, RPT), pl.ds(c * 64, 64)])

    @pl.when(on_c0)
    def _():
      pltpu.sync_copy(dacc.at[pl.ds(r0, RPT)], deg_out.at[pl.ds(r0, RPT)])

  return pl.kernel(body, out_type=out_type, mesh=mesh,
                   scratch_types=tuple(scratch),
                   compiler_params=pltpu.CompilerParams(
                       use_tc_tiling_on_sc=False))


def _segsum_cls_kernel():
  mesh = plsc.VectorSubcoreMesh(core_axis_name="c", subcore_axis_name="s")
  out_type = jax.ShapeDtypeStruct((NC, NPAD, CPAD), jnp.float32)
  scratch = [
      pltpu.VMEM((CPT_B, K), jnp.int32),
      pltpu.VMEM((CPT_B, K), jnp.int32),
      pltpu.VMEM((NB_B, K, CPAD), jnp.float32),
      pltpu.VMEM((ZB, CPAD), jnp.float32),
      pltpu.VMEM_SHARED((NPAD, CPAD), jnp.float32),
  ] + [pltpu.SemaphoreType.DMA] * (2 * NB_B)

  def body(feat, src, dst, out, sidx, didx, rows, zbuf, acc, *sems):
    gsem = sems[:NB_B]
    ssem = sems[NB_B:2 * NB_B]
    c = lax.axis_index("c")
    s = lax.axis_index("s")
    w = c * NS + s

    _fill(zbuf, ZB, CPAD, 0.0)

    pltpu.sync_copy(src.at[pl.ds(w * CPT_B, CPT_B)], sidx)
    pltpu.sync_copy(dst.at[pl.ds(w * CPT_B, CPT_B)], didx)

    r0 = s * RPT

    def zero_loop(i, _):
      pltpu.sync_copy(zbuf, acc.at[pl.ds(r0 + i * ZB, ZB)])
      return 0

    lax.fori_loop(0, RPT // ZB, zero_loop, 0)
    plsc.subcore_barrier()

    for b in range(PF_B):
      pltpu.async_copy(feat.at[sidx.at[b]], rows.at[b], gsem[b])

    def outer(g, _):
      for b in range(NB_B):
        cs = g * NB_B + b
        bg = (b + PF_B) % NB_B

        @pl.when(jnp.logical_and(cs >= NB_B - PF_B, cs < CPT_B - PF_B))
        def _():
          pltpu.make_async_copy(rows.at[bg], acc.at[didx.at[0]],
                                ssem[bg]).wait()

        @pl.when(cs < CPT_B - PF_B)
        def _():
          pltpu.async_copy(feat.at[sidx.at[cs + PF_B]], rows.at[bg],
                           gsem[bg])

        pltpu.make_async_copy(feat.at[sidx.at[cs]], rows.at[b],
                              gsem[b]).wait()
        pltpu.async_copy(rows.at[b], acc.at[didx.at[cs]], ssem[b], add=True)
      return 0

    lax.fori_loop(0, CPT_B // NB_B, outer, 0)

    for b in range(NB_B):
      pltpu.make_async_copy(rows.at[b], acc.at[didx.at[0]], ssem[b]).wait()
    plsc.subcore_barrier()

    pltpu.sync_copy(acc.at[pl.ds(r0, RPT)], out.at[c, pl.ds(r0, RPT)])

  return pl.kernel(body, out_type=out_type, mesh=mesh,
                   scratch_types=tuple(scratch),
                   compiler_params=pltpu.CompilerParams(
                       use_tc_tiling_on_sc=False))


_segsum_feat = _segsum_feat_kernel()
_segsum_cls = _segsum_cls_kernel()

BR = 1024


def _dense0_body(s0, degp, x, wl0, bl0, wr0, scale, shift, wl1, h_out, q_out):
  deg = degp[:, 0:1]
  mean = s0[...] / jnp.maximum(deg, 1.0)
  z = (jax.lax.dot(mean, wl0[...], preferred_element_type=jnp.float32)
       + bl0[...]
       + jax.lax.dot(x[...], wr0[...], preferred_element_type=jnp.float32))
  h = jnp.maximum(z * scale[...] + shift[...], 0.0)
  h_out[...] = h
  q_out[...] = jax.lax.dot(h, wl1[...], preferred_element_type=jnp.float32)


def _dense0(s0, degp, x, wl0, bl0, wr0, scale, shift, wl1):
  grid = (NPAD // BR,)
  return pl.pallas_call(
      _dense0_body,
      grid=grid,
      in_specs=[
          pl.BlockSpec((BR, NFEAT), lambda i: (i, 0)),
          pl.BlockSpec((BR, 16), lambda i: (i, 0)),
          pl.BlockSpec((BR, NFEAT), lambda i: (i, 0)),
          pl.BlockSpec((NFEAT, NHID), lambda i: (0, 0)),
          pl.BlockSpec((1, NHID), lambda i: (0, 0)),
          pl.BlockSpec((NFEAT, NHID), lambda i: (0, 0)),
          pl.BlockSpec((1, NHID), lambda i: (0, 0)),
          pl.BlockSpec((1, NHID), lambda i: (0, 0)),
          pl.BlockSpec((NHID, CPAD), lambda i: (0, 0)),
      ],
      out_specs=[
          pl.BlockSpec((BR, NHID), lambda i: (i, 0)),
          pl.BlockSpec((BR, CPAD), lambda i: (i, 0)),
      ],
      out_shape=[
          jax.ShapeDtypeStruct((NPAD, NHID), jnp.float32),
          jax.ShapeDtypeStruct((NPAD, CPAD), jnp.float32),
      ],
  )(s0, degp, x, wl0, bl0, wr0, scale, shift, wl1)


def _dense1_body(s1p, degp, h, wr1, bl1, out):
  ssum = s1p[0] + s1p[1]
  deg = degp[:, 0:1]
  z = (ssum / jnp.maximum(deg, 1.0) + bl1[...]
       + jax.lax.dot(h[...], wr1[...], preferred_element_type=jnp.float32))
  mask = lax.broadcasted_iota(jnp.int32, (1, CPAD), 1) < NCLASS
  z = jnp.where(mask, z, -1e30)
  m = jnp.max(z, axis=1, keepdims=True)
  ez = jnp.exp(z - m)
  lse = jnp.log(jnp.sum(ez, axis=1, keepdims=True))
  out[...] = z - m - lse


def _dense1(s1p, degp, h, wr1, bl1):
  grid = (NPAD // BR,)
  return pl.pallas_call(
      _dense1_body,
      grid=grid,
      in_specs=[
          pl.BlockSpec((NC, BR, CPAD), lambda i: (0, i, 0)),
          pl.BlockSpec((BR, 16), lambda i: (i, 0)),
          pl.BlockSpec((BR, NHID), lambda i: (i, 0)),
          pl.BlockSpec((NHID, CPAD), lambda i: (0, 0)),
          pl.BlockSpec((1, CPAD), lambda i: (0, 0)),
      ],
      out_specs=pl.BlockSpec((BR, CPAD), lambda i: (i, 0)),
      out_shape=jax.ShapeDtypeStruct((NPAD, CPAD), jnp.float32),
  )(s1p, degp, h, wr1, bl1)


def kernel(x, edge_index, W_l0, b_l0, W_r0, gamma0, beta0, W_l1, b_l1, W_r1):
  src = edge_index[0].reshape(E // K, K)
  dst = edge_index[1].reshape(E // K, K)
  x2 = x.reshape(2 * N, 64)
  s0, degp = _segsum_feat(x2, src, dst)

  scale = (gamma0 / jnp.sqrt(1.0 + BN_EPS)).reshape(1, NHID)
  shift = beta0.reshape(1, NHID)
  wl1 = jnp.pad(W_l1, ((0, 0), (0, CPAD - NCLASS)))
  xpad = jnp.pad(x, ((0, NPAD - N), (0, 0)))
  h, q = _dense0(s0, degp, xpad, W_l0, b_l0.reshape(1, NHID), W_r0,
                 scale, shift, wl1)

  s1p = _segsum_cls(q, src, dst)

  wr1 = jnp.pad(W_r1, ((0, 0), (0, CPAD - NCLASS)))
  bl1 = jnp.pad(b_l1, (0, CPAD - NCLASS)).reshape(1, CPAD)
  out = _dense1(s1p, degp, h, wr1, bl1)
  return out[:N, :NCLASS]


# speedup vs baseline: 3.1266x; 1.0002x over previous
import functools

import jax
import jax.numpy as jnp
from jax import lax
from jax.experimental import pallas as pl
from jax.experimental.pallas import tpu as pltpu
from jax.experimental.pallas import tpu_sc as plsc

N = 10000
NPAD = 10240
E = 320000
NFEAT = 128
NHID = 128
NCLASS = 47
CPAD = 48
BN_EPS = 1e-5

NC = 2
NS = 16
NW = NC * NS
K = 80
RPT = NPAD // NS
ZB = 40
NB_A = 5
NB_B = 5
PF_A = 4
PF_B = 4

CPT_A = E // NS // K
CPT_B = E // NW // K


def _fill(ref, rows, width, value):
  v = jnp.full((16,), value, ref.dtype)
  for r in range(rows):
    for j in range(width // 16):
      ref[r, pl.ds(j * 16, 16)] = v


def _segsum_feat_kernel():
  mesh = plsc.VectorSubcoreMesh(core_axis_name="c", subcore_axis_name="s")
  out_type = (jax.ShapeDtypeStruct((NPAD, NFEAT), jnp.float32),
              jax.ShapeDtypeStruct((NPAD, 16), jnp.float32))
  scratch = [
      pltpu.VMEM((CPT_A, K), jnp.int32),
      pltpu.VMEM((CPT_A, K), jnp.int32),
      pltpu.VMEM((NB_A, K, 64), jnp.float32),
      pltpu.VMEM((ZB, 64), jnp.float32),
      pltpu.VMEM((K, 16), jnp.float32),
      pltpu.VMEM((ZB, 16), jnp.float32),
      pltpu.VMEM_SHARED((NPAD, 64), jnp.float32),
      pltpu.VMEM_SHARED((NPAD, 16), jnp.float32),
  ] + [pltpu.SemaphoreType.DMA] * (2 * NB_A + 1)

  def body(feat2, src, dst, out, deg_out, sidx, didx, rows, zbuf, ones,
           dzbuf, acc, dacc, *sems):
    gsem = sems[:NB_A]
    ssem = sems[NB_A:2 * NB_A]
    dsem = sems[2 * NB_A]
    c = lax.axis_index("c")
    s = lax.axis_index("s")
    on_c0 = c == 0

    _fill(zbuf, ZB, 64, 0.0)
    _fill(ones, K, 16, 1.0)
    _fill(dzbuf, ZB, 16, 0.0)

    pltpu.sync_copy(src.at[pl.ds(s * CPT_A, CPT_A)], sidx)
    pltpu.sync_copy(dst.at[pl.ds(s * CPT_A, CPT_A)], didx)

    def xform(r, _):
      for j in range(K // 16):
        sl = pl.ds(j * 16, 16)
        sidx[r, sl] = sidx[r, sl] * 2 + c
      return 0

    lax.fori_loop(0, CPT_A, xform, 0)

    r0 = s * RPT

    def zero_loop(i, _):
      pltpu.sync_copy(zbuf, acc.at[pl.ds(r0 + i * ZB, ZB)])

      @pl.when(on_c0)
      def _():
        pltpu.sync_copy(dzbuf, dacc.at[pl.ds(r0 + i * ZB, ZB)])
      return 0

    lax.fori_loop(0, RPT // ZB, zero_loop, 0)
    plsc.subcore_barrier()

    for b in range(PF_A):
      pltpu.async_copy(feat2.at[sidx.at[b]], rows.at[b], gsem[b])

    def outer(g, _):
      for b in range(NB_A):
        cs = g * NB_A + b
        bg = (b + PF_A) % NB_A

        @pl.when(jnp.logical_and(cs >= NB_A - PF_A, cs < CPT_A - PF_A))
        def _():
          pltpu.make_async_copy(rows.at[bg], acc.at[didx.at[0]],
                                ssem[bg]).wait()

        @pl.when(cs < CPT_A - PF_A)
        def _():
          pltpu.async_copy(feat2.at[sidx.at[cs + PF_A]], rows.at[bg],
                           gsem[bg])

        pltpu.make_async_copy(feat2.at[sidx.at[cs]], rows.at[b],
                              gsem[b]).wait()
        pltpu.async_copy(rows.at[b], acc.at[didx.at[cs]], ssem[b], add=True)

        @pl.when(on_c0)
        def _():
          pltpu.async_copy(ones, dacc.at[didx.at[cs]], dsem, add=True)
      return 0

    lax.fori_loop(0, CPT_A // NB_A, outer, 0)

    for b in range(NB_A):
      pltpu.make_async_copy(rows.at[b], acc.at[didx.at[0]], ssem[b]).wait()

    @pl.when(on_c0)
    def _():
      def dloop(i, _):
        pltpu.make_async_copy(ones, dacc.at[didx.at[0]], dsem).wait()
        return 0
      lax.fori_loop(0, CPT_A, dloop, 0)

    plsc.subcore_barrier()

    pltpu.sync_copy(acc.at[pl.ds(r0, RPT)],
                    out.at[pl.ds(r0, RPT), pl.ds(c * 64, 64)])

    @pl.when(on_c0)
    def _():
      pltpu.sync_copy(dacc.at[pl.ds(r0, RPT)], deg_out.at[pl.ds(r0, RPT)])

  return pl.kernel(body, out_type=out_type, mesh=mesh,
                   scratch_types=tuple(scratch),
                   compiler_params=pltpu.CompilerParams(
                       use_tc_tiling_on_sc=False))


def _segsum_cls_kernel():
  mesh = plsc.VectorSubcoreMesh(core_axis_name="c", subcore_axis_name="s")
  out_type = jax.ShapeDtypeStruct((NC, NPAD, CPAD), jnp.float32)
  scratch = [
      pltpu.VMEM((CPT_B, K), jnp.int32),
      pltpu.VMEM((CPT_B, K), jnp.int32),
      pltpu.VMEM((NB_B, K, CPAD), jnp.float32),
      pltpu.VMEM((ZB, CPAD), jnp.float32),
      pltpu.VMEM_SHARED((NPAD, CPAD), jnp.float32),
  ] + [pltpu.SemaphoreType.DMA] * (2 * NB_B)

  def body(feat, src, dst, out, sidx, didx, rows, zbuf, acc, *sems):
    gsem = sems[:NB_B]
    ssem = sems[NB_B:2 * NB_B]
    c = lax.axis_index("c")
    s = lax.axis_index("s")
    w = c * NS + s

    _fill(zbuf, ZB, CPAD, 0.0)

    pltpu.sync_copy(src.at[pl.ds(w * CPT_B, CPT_B)], sidx)
    pltpu.sync_copy(dst.at[pl.ds(w * CPT_B, CPT_B)], didx)

    r0 = s * RPT

    def zero_loop(i, _):
      pltpu.sync_copy(zbuf, acc.at[pl.ds(r0 + i * ZB, ZB)])
      return 0

    lax.fori_loop(0, RPT // ZB, zero_loop, 0)
    plsc.subcore_barrier()

    for b in range(PF_B):
      pltpu.async_copy(feat.at[sidx.at[b]], rows.at[b], gsem[b])

    def outer(g, _):
      for b in range(NB_B):
        cs = g * NB_B + b
        bg = (b + PF_B) % NB_B

        @pl.when(jnp.logical_and(cs >= NB_B - PF_B, cs < CPT_B - PF_B))
        def _():
          pltpu.make_async_copy(rows.at[bg], acc.at[didx.at[0]],
                                ssem[bg]).wait()

        @pl.when(cs < CPT_B - PF_B)
        def _():
          pltpu.async_copy(feat.at[sidx.at[cs + PF_B]], rows.at[bg],
                           gsem[bg])

        pltpu.make_async_copy(feat.at[sidx.at[cs]], rows.at[b],
                              gsem[b]).wait()
        pltpu.async_copy(rows.at[b], acc.at[didx.at[cs]], ssem[b], add=True)
      return 0

    lax.fori_loop(0, CPT_B // NB_B, outer, 0)

    for b in range(NB_B):
      pltpu.make_async_copy(rows.at[b], acc.at[didx.at[0]], ssem[b]).wait()
    plsc.subcore_barrier()

    pltpu.sync_copy(acc.at[pl.ds(r0, RPT)], out.at[c, pl.ds(r0, RPT)])

  return pl.kernel(body, out_type=out_type, mesh=mesh,
                   scratch_types=tuple(scratch),
                   compiler_params=pltpu.CompilerParams(
                       use_tc_tiling_on_sc=False))


_segsum_feat = _segsum_feat_kernel()
_segsum_cls = _segsum_cls_kernel()

BR = 1024


def _bdot(a, b):
  return jax.lax.dot(a.astype(jnp.bfloat16), b.astype(jnp.bfloat16),
                     preferred_element_type=jnp.float32)


def _dense0_body(s0, degp, x, wl0, bl0, wr0, scale, shift, wl1, h_out, q_out):
  deg = degp[:, 0:1]
  mean = s0[...] / jnp.maximum(deg, 1.0)
  z = _bdot(mean, wl0[...]) + bl0[...] + _bdot(x[...], wr0[...])
  h = jnp.maximum(z * scale[...] + shift[...], 0.0)
  h_out[...] = h
  q_out[...] = _bdot(h, wl1[...])


def _dense0(s0, degp, x, wl0, bl0, wr0, scale, shift, wl1):
  grid = (NPAD // BR,)
  return pl.pallas_call(
      _dense0_body,
      grid=grid,
      in_specs=[
          pl.BlockSpec((BR, NFEAT), lambda i: (i, 0)),
          pl.BlockSpec((BR, 16), lambda i: (i, 0)),
          pl.BlockSpec((BR, NFEAT), lambda i: (i, 0)),
          pl.BlockSpec((NFEAT, NHID), lambda i: (0, 0)),
          pl.BlockSpec((1, NHID), lambda i: (0, 0)),
          pl.BlockSpec((NFEAT, NHID), lambda i: (0, 0)),
          pl.BlockSpec((1, NHID), lambda i: (0, 0)),
          pl.BlockSpec((1, NHID), lambda i: (0, 0)),
          pl.BlockSpec((NHID, CPAD), lambda i: (0, 0)),
      ],
      out_specs=[
          pl.BlockSpec((BR, NHID), lambda i: (i, 0)),
          pl.BlockSpec((BR, CPAD), lambda i: (i, 0)),
      ],
      out_shape=[
          jax.ShapeDtypeStruct((NPAD, NHID), jnp.float32),
          jax.ShapeDtypeStruct((NPAD, CPAD), jnp.float32),
      ],
  )(s0, degp, x, wl0, bl0, wr0, scale, shift, wl1)


def _dense1_body(s1p, degp, h, wr1, bl1, out):
  ssum = s1p[0] + s1p[1]
  deg = degp[:, 0:1]
  z = ssum / jnp.maximum(deg, 1.0) + bl1[...] + _bdot(h[...], wr1[...])
  mask = lax.broadcasted_iota(jnp.int32, (1, CPAD), 1) < NCLASS
  z = jnp.where(mask, z, -1e30)
  m = jnp.max(z, axis=1, keepdims=True)
  ez = jnp.exp(z - m)
  lse = jnp.log(jnp.sum(ez, axis=1, keepdims=True))
  out[...] = z - m - lse


def _dense1(s1p, degp, h, wr1, bl1):
  grid = (NPAD // BR,)
  return pl.pallas_call(
      _dense1_body,
      grid=grid,
      in_specs=[
          pl.BlockSpec((NC, BR, CPAD), lambda i: (0, i, 0)),
          pl.BlockSpec((BR, 16), lambda i: (i, 0)),
          pl.BlockSpec((BR, NHID), lambda i: (i, 0)),
          pl.BlockSpec((NHID, CPAD), lambda i: (0, 0)),
          pl.BlockSpec((1, CPAD), lambda i: (0, 0)),
      ],
      out_specs=pl.BlockSpec((BR, CPAD), lambda i: (i, 0)),
      out_shape=jax.ShapeDtypeStruct((NPAD, CPAD), jnp.float32),
  )(s1p, degp, h, wr1, bl1)


def kernel(x, edge_index, W_l0, b_l0, W_r0, gamma0, beta0, W_l1, b_l1, W_r1):
  src = edge_index[0].reshape(E // K, K)
  dst = edge_index[1].reshape(E // K, K)
  x2 = x.reshape(2 * N, 64)
  s0, degp = _segsum_feat(x2, src, dst)

  scale = (gamma0 / jnp.sqrt(1.0 + BN_EPS)).reshape(1, NHID)
  shift = beta0.reshape(1, NHID)
  wl1 = jnp.pad(W_l1, ((0, 0), (0, CPAD - NCLASS)))
  xpad = jnp.pad(x, ((0, NPAD - N), (0, 0)))
  h, q = _dense0(s0, degp, xpad, W_l0, b_l0.reshape(1, NHID), W_r0,
                 scale, shift, wl1)

  s1p = _segsum_cls(q, src, dst)

  wr1 = jnp.pad(W_r1, ((0, 0), (0, CPAD - NCLASS)))
  bl1 = jnp.pad(b_l1, (0, CPAD - NCLASS)).reshape(1, CPAD)
  out = _dense1(s1p, degp, h, wr1, bl1)
  return out[:N, :NCLASS]


# speedup vs baseline: 3.1716x; 1.0144x over previous
import functools

import jax
import jax.numpy as jnp
from jax import lax
from jax.experimental import pallas as pl
from jax.experimental.pallas import tpu as pltpu
from jax.experimental.pallas import tpu_sc as plsc

N = 10000
NPAD = 10240
E = 320000
NFEAT = 128
NHID = 128
NCLASS = 47
CPAD = 48
BN_EPS = 1e-5

NC = 2
NS = 16
NW = NC * NS
K = 80
RPT = NPAD // NS
ZB = 40
NB_A = 5
NB_B = 5
PF_A = 4
PF_B = 4

CPT_A = E // NS // K
CPT_B = E // NW // K


def _fill(ref, rows, width, value):
  v = jnp.full((16,), value, ref.dtype)
  for r in range(rows):
    for j in range(width // 16):
      ref[r, pl.ds(j * 16, 16)] = v


def _segsum_feat_kernel():
  mesh = plsc.VectorSubcoreMesh(core_axis_name="c", subcore_axis_name="s")
  out_type = (jax.ShapeDtypeStruct((NPAD, NFEAT), jnp.float32),
              jax.ShapeDtypeStruct((NPAD, 16), jnp.float32))
  scratch = [
      pltpu.VMEM((CPT_A, K), jnp.int32),
      pltpu.VMEM((CPT_A, K), jnp.int32),
      pltpu.VMEM((NB_A, K, 64), jnp.float32),
      pltpu.VMEM((ZB, 64), jnp.float32),
      pltpu.VMEM((K, 16), jnp.float32),
      pltpu.VMEM((ZB, 16), jnp.float32),
      pltpu.VMEM_SHARED((NPAD, 64), jnp.float32),
      pltpu.VMEM_SHARED((NPAD, 16), jnp.float32),
  ] + [pltpu.SemaphoreType.DMA] * (2 * NB_A + 1)

  def body(feat2, src, dst, out, deg_out, sidx, didx, rows, zbuf, ones,
           dzbuf, acc, dacc, *sems):
    gsem = sems[:NB_A]
    ssem = sems[NB_A:2 * NB_A]
    dsem = sems[2 * NB_A]
    c = lax.axis_index("c")
    s = lax.axis_index("s")
    on_c0 = c == 0

    _fill(zbuf, ZB, 64, 0.0)
    _fill(ones, K, 16, 1.0)
    _fill(dzbuf, ZB, 16, 0.0)

    pltpu.sync_copy(src.at[pl.ds(s * CPT_A, CPT_A)], sidx)
    pltpu.sync_copy(dst.at[pl.ds(s * CPT_A, CPT_A)], didx)

    def xform(r, _):
      for j in range(K // 16):
        sl = pl.ds(j * 16, 16)
        sidx[r, sl] = sidx[r, sl] * 2 + c
      return 0

    lax.fori_loop(0, CPT_A, xform, 0)

    r0 = s * RPT

    def zero_loop(i, _):
      pltpu.sync_copy(zbuf, acc.at[pl.ds(r0 + i * ZB, ZB)])

      @pl.when(on_c0)
      def _():
        pltpu.sync_copy(dzbuf, dacc.at[pl.ds(r0 + i * ZB, ZB)])
      return 0

    lax.fori_loop(0, RPT // ZB, zero_loop, 0)
    plsc.subcore_barrier()

    for b in range(PF_A):
      pltpu.async_copy(feat2.at[sidx.at[b]], rows.at[b], gsem[b])

    def outer(g, _):
      for b in range(NB_A):
        cs = g * NB_A + b
        bg = (b + PF_A) % NB_A

        @pl.when(jnp.logical_and(cs >= NB_A - PF_A, cs < CPT_A - PF_A))
        def _():
          pltpu.make_async_copy(rows.at[bg], acc.at[didx.at[0]],
                                ssem[bg]).wait()

        @pl.when(cs < CPT_A - PF_A)
        def _():
          pltpu.async_copy(feat2.at[sidx.at[cs + PF_A]], rows.at[bg],
                           gsem[bg])

        pltpu.make_async_copy(feat2.at[sidx.at[cs]], rows.at[b],
                              gsem[b]).wait()
        pltpu.async_copy(rows.at[b], acc.at[didx.at[cs]], ssem[b], add=True)

        @pl.when(on_c0)
        def _():
          pltpu.async_copy(ones, dacc.at[didx.at[cs]], dsem, add=True)
      return 0

    lax.fori_loop(0, CPT_A // NB_A, outer, 0)

    for b in range(NB_A):
      pltpu.make_async_copy(rows.at[b], acc.at[didx.at[0]], ssem[b]).wait()

    @pl.when(on_c0)
    def _():
      def dloop(i, _):
        pltpu.make_async_copy(ones, dacc.at[didx.at[0]], dsem).wait()
        return 0
      lax.fori_loop(0, CPT_A, dloop, 0)

    plsc.subcore_barrier()

    pltpu.sync_copy(acc.at[pl.ds(r0, RPT)],
                    out.at[pl.ds(r0, RPT), pl.ds(c * 64, 64)])

    @pl.when(on_c0)
    def _():
      pltpu.sync_copy(dacc.at[pl.ds(r0, RPT)], deg_out.at[pl.ds(r0, RPT)])

  return pl.kernel(body, out_type=out_type, mesh=mesh,
                   scratch_types=tuple(scratch),
                   compiler_params=pltpu.CompilerParams(
                       use_tc_tiling_on_sc=False))


def _segsum_cls_kernel():
  mesh = plsc.VectorSubcoreMesh(core_axis_name="c", subcore_axis_name="s")
  out_type = jax.ShapeDtypeStruct((NC, NPAD, CPAD), jnp.float32)
  scratch = [
      pltpu.VMEM((CPT_B, K), jnp.int32),
      pltpu.VMEM((CPT_B, K), jnp.int32),
      pltpu.VMEM((NB_B, K, CPAD), jnp.float32),
      pltpu.VMEM((ZB, CPAD), jnp.float32),
      pltpu.VMEM_SHARED((NPAD, CPAD), jnp.float32),
  ] + [pltpu.SemaphoreType.DMA] * (2 * NB_B)

  def body(feat, src, dst, out, sidx, didx, rows, zbuf, acc, *sems):
    gsem = sems[:NB_B]
    ssem = sems[NB_B:2 * NB_B]
    c = lax.axis_index("c")
    s = lax.axis_index("s")
    w = c * NS + s

    _fill(zbuf, ZB, CPAD, 0.0)

    pltpu.sync_copy(src.at[pl.ds(w * CPT_B, CPT_B)], sidx)
    pltpu.sync_copy(dst.at[pl.ds(w * CPT_B, CPT_B)], didx)

    r0 = s * RPT

    def zero_loop(i, _):
      pltpu.sync_copy(zbuf, acc.at[pl.ds(r0 + i * ZB, ZB)])
      return 0

    lax.fori_loop(0, RPT // ZB, zero_loop, 0)
    plsc.subcore_barrier()

    for b in range(PF_B):
      pltpu.async_copy(feat.at[sidx.at[b]], rows.at[b], gsem[b])

    def outer(g, _):
      for b in range(NB_B):
        cs = g * NB_B + b
        bg = (b + PF_B) % NB_B

        @pl.when(jnp.logical_and(cs >= NB_B - PF_B, cs < CPT_B - PF_B))
        def _():
          pltpu.make_async_copy(rows.at[bg], acc.at[didx.at[0]],
                                ssem[bg]).wait()

        @pl.when(cs < CPT_B - PF_B)
        def _():
          pltpu.async_copy(feat.at[sidx.at[cs + PF_B]], rows.at[bg],
                           gsem[bg])

        pltpu.make_async_copy(feat.at[sidx.at[cs]], rows.at[b],
                              gsem[b]).wait()
        pltpu.async_copy(rows.at[b], acc.at[didx.at[cs]], ssem[b], add=True)
      return 0

    lax.fori_loop(0, CPT_B // NB_B, outer, 0)

    for b in range(NB_B):
      pltpu.make_async_copy(rows.at[b], acc.at[didx.at[0]], ssem[b]).wait()
    plsc.subcore_barrier()

    pltpu.sync_copy(acc.at[pl.ds(r0, RPT)], out.at[c, pl.ds(r0, RPT)])

  return pl.kernel(body, out_type=out_type, mesh=mesh,
                   scratch_types=tuple(scratch),
                   compiler_params=pltpu.CompilerParams(
                       use_tc_tiling_on_sc=False))


_segsum_feat = _segsum_feat_kernel()
_segsum_cls = _segsum_cls_kernel()

BR = 2048


def _bdot(a, b):
  return jax.lax.dot(a.astype(jnp.bfloat16), b.astype(jnp.bfloat16),
                     preferred_element_type=jnp.float32)


def _dense0_body(s0, degp, x, wl0, bl0, wr0, scale, shift, wl1, h_out, q_out):
  deg = degp[:, 0:1]
  mean = s0[...] / jnp.maximum(deg, 1.0)
  z = _bdot(mean, wl0[...]) + bl0[...] + _bdot(x[...], wr0[...])
  h = jnp.maximum(z * scale[...] + shift[...], 0.0)
  h_out[...] = h
  q_out[...] = _bdot(h, wl1[...])


def _dense0(s0, degp, x, wl0, bl0, wr0, scale, shift, wl1):
  grid = (NPAD // BR,)
  return pl.pallas_call(
      _dense0_body,
      grid=grid,
      in_specs=[
          pl.BlockSpec((BR, NFEAT), lambda i: (i, 0)),
          pl.BlockSpec((BR, 16), lambda i: (i, 0)),
          pl.BlockSpec((BR, NFEAT), lambda i: (i, 0)),
          pl.BlockSpec((NFEAT, NHID), lambda i: (0, 0)),
          pl.BlockSpec((1, NHID), lambda i: (0, 0)),
          pl.BlockSpec((NFEAT, NHID), lambda i: (0, 0)),
          pl.BlockSpec((1, NHID), lambda i: (0, 0)),
          pl.BlockSpec((1, NHID), lambda i: (0, 0)),
          pl.BlockSpec((NHID, CPAD), lambda i: (0, 0)),
      ],
      out_specs=[
          pl.BlockSpec((BR, NHID), lambda i: (i, 0)),
          pl.BlockSpec((BR, CPAD), lambda i: (i, 0)),
      ],
      out_shape=[
          jax.ShapeDtypeStruct((NPAD, NHID), jnp.float32),
          jax.ShapeDtypeStruct((NPAD, CPAD), jnp.float32),
      ],
  )(s0, degp, x, wl0, bl0, wr0, scale, shift, wl1)


def _dense1_body(s1p, degp, h, wr1, bl1, out):
  ssum = s1p[0] + s1p[1]
  deg = degp[:, 0:1]
  z = ssum / jnp.maximum(deg, 1.0) + bl1[...] + _bdot(h[...], wr1[...])
  mask = lax.broadcasted_iota(jnp.int32, (1, CPAD), 1) < NCLASS
  z = jnp.where(mask, z, -1e30)
  m = jnp.max(z, axis=1, keepdims=True)
  ez = jnp.exp(z - m)
  lse = jnp.log(jnp.sum(ez, axis=1, keepdims=True))
  out[...] = z - m - lse


def _dense1(s1p, degp, h, wr1, bl1):
  grid = (NPAD // BR,)
  return pl.pallas_call(
      _dense1_body,
      grid=grid,
      in_specs=[
          pl.BlockSpec((NC, BR, CPAD), lambda i: (0, i, 0)),
          pl.BlockSpec((BR, 16), lambda i: (i, 0)),
          pl.BlockSpec((BR, NHID), lambda i: (i, 0)),
          pl.BlockSpec((NHID, CPAD), lambda i: (0, 0)),
          pl.BlockSpec((1, CPAD), lambda i: (0, 0)),
      ],
      out_specs=pl.BlockSpec((BR, CPAD), lambda i: (i, 0)),
      out_shape=jax.ShapeDtypeStruct((NPAD, CPAD), jnp.float32),
  )(s1p, degp, h, wr1, bl1)


def kernel(x, edge_index, W_l0, b_l0, W_r0, gamma0, beta0, W_l1, b_l1, W_r1):
  src = edge_index[0].reshape(E // K, K)
  dst = edge_index[1].reshape(E // K, K)
  x2 = x.reshape(2 * N, 64)
  s0, degp = _segsum_feat(x2, src, dst)

  scale = (gamma0 / jnp.sqrt(1.0 + BN_EPS)).reshape(1, NHID)
  shift = beta0.reshape(1, NHID)
  wl1 = jnp.pad(W_l1, ((0, 0), (0, CPAD - NCLASS)))
  xpad = jnp.pad(x, ((0, NPAD - N), (0, 0)))
  h, q = _dense0(s0, degp, xpad, W_l0, b_l0.reshape(1, NHID), W_r0,
                 scale, shift, wl1)

  s1p = _segsum_cls(q, src, dst)

  wr1 = jnp.pad(W_r1, ((0, 0), (0, CPAD - NCLASS)))
  bl1 = jnp.pad(b_l1, (0, CPAD - NCLASS)).reshape(1, CPAD)
  out = _dense1(s1p, degp, h, wr1, bl1)
  return out[:N, :NCLASS]


# speedup vs baseline: 3.2298x; 1.0184x over previous
import functools

import jax
import jax.numpy as jnp
from jax import lax
from jax.experimental import pallas as pl
from jax.experimental.pallas import tpu as pltpu
from jax.experimental.pallas import tpu_sc as plsc

N = 10000
NPAD = 10240
E = 320000
NFEAT = 128
NHID = 128
NCLASS = 47
CPAD = 48
BN_EPS = 1e-5

NC = 2
NS = 16
NW = NC * NS
K = 80
RPT = NPAD // NS
ZB = 40
NB_A = 5
NB_B = 5
PF_A = 4
PF_B = 4

CPT_A = E // NS // K
CPT_B = E // NW // K


def _fill(ref, rows, width, value):
  v = jnp.full((16,), value, ref.dtype)
  for r in range(rows):
    for j in range(width // 16):
      ref[r, pl.ds(j * 16, 16)] = v


def _segsum_feat_kernel():
  mesh = plsc.VectorSubcoreMesh(core_axis_name="c", subcore_axis_name="s")
  out_type = (jax.ShapeDtypeStruct((NPAD, NFEAT), jnp.float32),
              jax.ShapeDtypeStruct((NPAD, 16), jnp.float32))
  scratch = [
      pltpu.VMEM((CPT_A, K), jnp.int32),
      pltpu.VMEM((CPT_A, K), jnp.int32),
      pltpu.VMEM((NB_A, K, 64), jnp.float32),
      pltpu.VMEM((ZB, 64), jnp.float32),
      pltpu.VMEM((K, 16), jnp.float32),
      pltpu.VMEM((ZB, 16), jnp.float32),
      pltpu.VMEM_SHARED((NPAD, 64), jnp.float32),
      pltpu.VMEM_SHARED((NPAD, 16), jnp.float32),
  ] + [pltpu.SemaphoreType.DMA] * (2 * NB_A + 1)

  def body(feat2, src, dst, out, deg_out, sidx, didx, rows, zbuf, ones,
           dzbuf, acc, dacc, *sems):
    gsem = sems[:NB_A]
    ssem = sems[NB_A:2 * NB_A]
    dsem = sems[2 * NB_A]
    c = lax.axis_index("c")
    s = lax.axis_index("s")
    on_c0 = c == 0

    _fill(zbuf, ZB, 64, 0.0)
    _fill(ones, K, 16, 1.0)
    _fill(dzbuf, ZB, 16, 0.0)

    pltpu.sync_copy(src.at[pl.ds(s * CPT_A, CPT_A)], sidx)
    pltpu.sync_copy(dst.at[pl.ds(s * CPT_A, CPT_A)], didx)

    def xform(r, _):
      for j in range(K // 16):
        sl = pl.ds(j * 16, 16)
        sidx[r, sl] = sidx[r, sl] * 2 + c
      return 0

    lax.fori_loop(0, CPT_A, xform, 0)

    r0 = s * RPT

    def zero_loop(i, _):
      pltpu.sync_copy(zbuf, acc.at[pl.ds(r0 + i * ZB, ZB)])

      @pl.when(on_c0)
      def _():
        pltpu.sync_copy(dzbuf, dacc.at[pl.ds(r0 + i * ZB, ZB)])
      return 0

    lax.fori_loop(0, RPT // ZB, zero_loop, 0)
    plsc.subcore_barrier()

    for b in range(PF_A):
      pltpu.async_copy(feat2.at[sidx.at[b]], rows.at[b], gsem[b])

    def outer(g, _):
      for b in range(NB_A):
        cs = g * NB_A + b
        bg = (b + PF_A) % NB_A

        @pl.when(jnp.logical_and(cs >= NB_A - PF_A, cs < CPT_A - PF_A))
        def _():
          pltpu.make_async_copy(rows.at[bg], acc.at[didx.at[0]],
                                ssem[bg]).wait()

        @pl.when(cs < CPT_A - PF_A)
        def _():
          pltpu.async_copy(feat2.at[sidx.at[cs + PF_A]], rows.at[bg],
                           gsem[bg])

        pltpu.make_async_copy(feat2.at[sidx.at[cs]], rows.at[b],
                              gsem[b]).wait()
        pltpu.async_copy(rows.at[b], acc.at[didx.at[cs]], ssem[b], add=True)

        @pl.when(on_c0)
        def _():
          pltpu.async_copy(ones, dacc.at[didx.at[cs]], dsem, add=True)
      return 0

    lax.fori_loop(0, CPT_A // NB_A, outer, 0)

    for b in range(NB_A):
      pltpu.make_async_copy(rows.at[b], acc.at[didx.at[0]], ssem[b]).wait()

    @pl.when(on_c0)
    def _():
      def dloop(i, _):
        pltpu.make_async_copy(ones, dacc.at[didx.at[0]], dsem).wait()
        return 0
      lax.fori_loop(0, CPT_A, dloop, 0)

    plsc.subcore_barrier()

    pltpu.sync_copy(acc.at[pl.ds(r0, RPT)],
                    out.at[pl.ds(r0, RPT), pl.ds(c * 64, 64)])

    @pl.when(on_c0)
    def _():
      pltpu.sync_copy(dacc.at[pl.ds(r0, RPT)], deg_out.at[pl.ds(r0, RPT)])

  return pl.kernel(body, out_type=out_type, mesh=mesh,
                   scratch_types=tuple(scratch),
                   compiler_params=pltpu.CompilerParams(
                       use_tc_tiling_on_sc=False))


def _segsum_cls_kernel():
  mesh = plsc.VectorSubcoreMesh(core_axis_name="c", subcore_axis_name="s")
  out_type = jax.ShapeDtypeStruct((NC, NPAD, CPAD), jnp.float32)
  scratch = [
      pltpu.VMEM((CPT_B, K), jnp.int32),
      pltpu.VMEM((CPT_B, K), jnp.int32),
      pltpu.VMEM((NB_B, K, CPAD), jnp.float32),
      pltpu.VMEM((ZB, CPAD), jnp.float32),
      pltpu.VMEM_SHARED((NPAD, CPAD), jnp.float32),
  ] + [pltpu.SemaphoreType.DMA] * (2 * NB_B)

  def body(feat, src, dst, out, sidx, didx, rows, zbuf, acc, *sems):
    gsem = sems[:NB_B]
    ssem = sems[NB_B:2 * NB_B]
    c = lax.axis_index("c")
    s = lax.axis_index("s")
    w = c * NS + s

    _fill(zbuf, ZB, CPAD, 0.0)

    pltpu.sync_copy(src.at[pl.ds(w * CPT_B, CPT_B)], sidx)
    pltpu.sync_copy(dst.at[pl.ds(w * CPT_B, CPT_B)], didx)

    r0 = s * RPT

    def zero_loop(i, _):
      pltpu.sync_copy(zbuf, acc.at[pl.ds(r0 + i * ZB, ZB)])
      return 0

    lax.fori_loop(0, RPT // ZB, zero_loop, 0)
    plsc.subcore_barrier()

    for b in range(PF_B):
      pltpu.async_copy(feat.at[sidx.at[b]], rows.at[b], gsem[b])

    def outer(g, _):
      for b in range(NB_B):
        cs = g * NB_B + b
        bg = (b + PF_B) % NB_B

        @pl.when(jnp.logical_and(cs >= NB_B - PF_B, cs < CPT_B - PF_B))
        def _():
          pltpu.make_async_copy(rows.at[bg], acc.at[didx.at[0]],
                                ssem[bg]).wait()

        @pl.when(cs < CPT_B - PF_B)
        def _():
          pltpu.async_copy(feat.at[sidx.at[cs + PF_B]], rows.at[bg],
                           gsem[bg])

        pltpu.make_async_copy(feat.at[sidx.at[cs]], rows.at[b],
                              gsem[b]).wait()
        pltpu.async_copy(rows.at[b], acc.at[didx.at[cs]], ssem[b], add=True)
      return 0

    lax.fori_loop(0, CPT_B // NB_B, outer, 0)

    for b in range(NB_B):
      pltpu.make_async_copy(rows.at[b], acc.at[didx.at[0]], ssem[b]).wait()
    plsc.subcore_barrier()

    pltpu.sync_copy(acc.at[pl.ds(r0, RPT)], out.at[c, pl.ds(r0, RPT)])

  return pl.kernel(body, out_type=out_type, mesh=mesh,
                   scratch_types=tuple(scratch),
                   compiler_params=pltpu.CompilerParams(
                       use_tc_tiling_on_sc=False))


_segsum_feat = _segsum_feat_kernel()
_segsum_cls = _segsum_cls_kernel()

BR = 5120


def _bdot(a, b):
  return jax.lax.dot(a.astype(jnp.bfloat16), b.astype(jnp.bfloat16),
                     preferred_element_type=jnp.float32)


def _dense0_body(s0, degp, x, wl0, bl0, wr0, scale, shift, wl1, h_out, q_out):
  deg = degp[:, 0:1]
  mean = s0[...] / jnp.maximum(deg, 1.0)
  z = _bdot(mean, wl0[...]) + bl0[...] + _bdot(x[...], wr0[...])
  h = jnp.maximum(z * scale[...] + shift[...], 0.0)
  h_out[...] = h
  q_out[...] = _bdot(h, wl1[...])


def _dense0(s0, degp, x, wl0, bl0, wr0, scale, shift, wl1):
  grid = (NPAD // BR,)
  return pl.pallas_call(
      _dense0_body,
      grid=grid,
      in_specs=[
          pl.BlockSpec((BR, NFEAT), lambda i: (i, 0)),
          pl.BlockSpec((BR, 16), lambda i: (i, 0)),
          pl.BlockSpec((BR, NFEAT), lambda i: (i, 0)),
          pl.BlockSpec((NFEAT, NHID), lambda i: (0, 0)),
          pl.BlockSpec((1, NHID), lambda i: (0, 0)),
          pl.BlockSpec((NFEAT, NHID), lambda i: (0, 0)),
          pl.BlockSpec((1, NHID), lambda i: (0, 0)),
          pl.BlockSpec((1, NHID), lambda i: (0, 0)),
          pl.BlockSpec((NHID, CPAD), lambda i: (0, 0)),
      ],
      out_specs=[
          pl.BlockSpec((BR, NHID), lambda i: (i, 0)),
          pl.BlockSpec((BR, CPAD), lambda i: (i, 0)),
      ],
      out_shape=[
          jax.ShapeDtypeStruct((NPAD, NHID), jnp.float32),
          jax.ShapeDtypeStruct((NPAD, CPAD), jnp.float32),
      ],
  )(s0, degp, x, wl0, bl0, wr0, scale, shift, wl1)


def _dense1_body(s1p, degp, h, wr1, bl1, out):
  ssum = s1p[0] + s1p[1]
  deg = degp[:, 0:1]
  z = ssum / jnp.maximum(deg, 1.0) + bl1[...] + _bdot(h[...], wr1[...])
  mask = lax.broadcasted_iota(jnp.int32, (1, CPAD), 1) < NCLASS
  z = jnp.where(mask, z, -1e30)
  m = jnp.max(z, axis=1, keepdims=True)
  ez = jnp.exp(z - m)
  lse = jnp.log(jnp.sum(ez, axis=1, keepdims=True))
  out[...] = z - m - lse


def _dense1(s1p, degp, h, wr1, bl1):
  grid = (NPAD // BR,)
  return pl.pallas_call(
      _dense1_body,
      grid=grid,
      in_specs=[
          pl.BlockSpec((NC, BR, CPAD), lambda i: (0, i, 0)),
          pl.BlockSpec((BR, 16), lambda i: (i, 0)),
          pl.BlockSpec((BR, NHID), lambda i: (i, 0)),
          pl.BlockSpec((NHID, CPAD), lambda i: (0, 0)),
          pl.BlockSpec((1, CPAD), lambda i: (0, 0)),
      ],
      out_specs=pl.BlockSpec((BR, CPAD), lambda i: (i, 0)),
      out_shape=jax.ShapeDtypeStruct((NPAD, CPAD), jnp.float32),
  )(s1p, degp, h, wr1, bl1)


def kernel(x, edge_index, W_l0, b_l0, W_r0, gamma0, beta0, W_l1, b_l1, W_r1):
  src = edge_index[0].reshape(E // K, K)
  dst = edge_index[1].reshape(E // K, K)
  x2 = x.reshape(2 * N, 64)
  s0, degp = _segsum_feat(x2, src, dst)

  scale = (gamma0 / jnp.sqrt(1.0 + BN_EPS)).reshape(1, NHID)
  shift = beta0.reshape(1, NHID)
  wl1 = jnp.pad(W_l1, ((0, 0), (0, CPAD - NCLASS)))
  xpad = jnp.pad(x, ((0, NPAD - N), (0, 0)))
  h, q = _dense0(s0, degp, xpad, W_l0, b_l0.reshape(1, NHID), W_r0,
                 scale, shift, wl1)

  s1p = _segsum_cls(q, src, dst)

  wr1 = jnp.pad(W_r1, ((0, 0), (0, CPAD - NCLASS)))
  bl1 = jnp.pad(b_l1, (0, CPAD - NCLASS)).reshape(1, CPAD)
  out = _dense1(s1p, degp, h, wr1, bl1)
  return out[:N, :NCLASS]


# speedup vs baseline: 3.2381x; 1.0026x over previous
import functools

import jax
import jax.numpy as jnp
from jax import lax
from jax.experimental import pallas as pl
from jax.experimental.pallas import tpu as pltpu
from jax.experimental.pallas import tpu_sc as plsc

N = 10000
NPAD = 10240
E = 320000
NFEAT = 128
NHID = 128
NCLASS = 47
CPAD = 48
BN_EPS = 1e-5

NC = 2
NS = 16
NW = NC * NS
K = 80
RPT = NPAD // NS
ZB = 40
NB_A = 5
NB_B = 5
PF_A = 4
PF_B = 4

CPT_A = E // NS // K
CPT_B = E // NW // K


def _fill(ref, rows, width, value):
  v = jnp.full((16,), value, ref.dtype)
  for r in range(rows):
    for j in range(width // 16):
      ref[r, pl.ds(j * 16, 16)] = v


def _segsum_feat_kernel():
  mesh = plsc.VectorSubcoreMesh(core_axis_name="c", subcore_axis_name="s")
  out_type = (jax.ShapeDtypeStruct((NPAD, NFEAT), jnp.float32),
              jax.ShapeDtypeStruct((NPAD, 16), jnp.float32))
  scratch = [
      pltpu.VMEM((CPT_A, K), jnp.int32),
      pltpu.VMEM((CPT_A, K), jnp.int32),
      pltpu.VMEM((NB_A, K, 64), jnp.float32),
      pltpu.VMEM((ZB, 64), jnp.float32),
      pltpu.VMEM((K, 16), jnp.float32),
      pltpu.VMEM((ZB, 16), jnp.float32),
      pltpu.VMEM_SHARED((NPAD, 64), jnp.float32),
      pltpu.VMEM_SHARED((NPAD, 16), jnp.float32),
  ] + [pltpu.SemaphoreType.DMA] * (2 * NB_A + 1)

  def body(feat2, src, dst, out, deg_out, sidx, didx, rows, zbuf, ones,
           dzbuf, acc, dacc, *sems):
    gsem = sems[:NB_A]
    ssem = sems[NB_A:2 * NB_A]
    dsem = sems[2 * NB_A]
    c = lax.axis_index("c")
    s = lax.axis_index("s")
    on_c0 = c == 0

    _fill(zbuf, ZB, 64, 0.0)
    _fill(ones, K, 16, 1.0)
    _fill(dzbuf, ZB, 16, 0.0)

    pltpu.sync_copy(src.at[pl.ds(s * CPT_A, CPT_A)], sidx)
    pltpu.sync_copy(dst.at[pl.ds(s * CPT_A, CPT_A)], didx)

    def xform(r, _):
      for j in range(K // 16):
        sl = pl.ds(j * 16, 16)
        sidx[r, sl] = sidx[r, sl] * 2 + c
      return 0

    lax.fori_loop(0, CPT_A, xform, 0)

    r0 = s * RPT

    def zero_loop(i, _):
      pltpu.sync_copy(zbuf, acc.at[pl.ds(r0 + i * ZB, ZB)])

      @pl.when(on_c0)
      def _():
        pltpu.sync_copy(dzbuf, dacc.at[pl.ds(r0 + i * ZB, ZB)])
      return 0

    lax.fori_loop(0, RPT // ZB, zero_loop, 0)
    plsc.subcore_barrier()

    for b in range(PF_A):
      pltpu.async_copy(feat2.at[sidx.at[b]], rows.at[b], gsem[b])

    def outer(g, _):
      for b in range(NB_A):
        cs = g * NB_A + b
        bg = (b + PF_A) % NB_A

        @pl.when(jnp.logical_and(cs >= NB_A - PF_A, cs < CPT_A - PF_A))
        def _():
          pltpu.make_async_copy(rows.at[bg], acc.at[didx.at[0]],
                                ssem[bg]).wait()

        @pl.when(cs < CPT_A - PF_A)
        def _():
          pltpu.async_copy(feat2.at[sidx.at[cs + PF_A]], rows.at[bg],
                           gsem[bg])

        pltpu.make_async_copy(feat2.at[sidx.at[cs]], rows.at[b],
                              gsem[b]).wait()
        pltpu.async_copy(rows.at[b], acc.at[didx.at[cs]], ssem[b], add=True)

        @pl.when(on_c0)
        def _():
          pltpu.async_copy(ones, dacc.at[didx.at[cs]], dsem, add=True)
      return 0

    lax.fori_loop(0, CPT_A // NB_A, outer, 0)

    for b in range(NB_A):
      pltpu.make_async_copy(rows.at[b], acc.at[didx.at[0]], ssem[b]).wait()

    @pl.when(on_c0)
    def _():
      def dloop(i, _):
        pltpu.make_async_copy(ones, dacc.at[didx.at[0]], dsem).wait()
        return 0
      lax.fori_loop(0, CPT_A, dloop, 0)

    plsc.subcore_barrier()

    pltpu.sync_copy(acc.at[pl.ds(r0, RPT)],
                    out.at[pl.ds(r0, RPT), pl.ds(c * 64, 64)])

    @pl.when(on_c0)
    def _():
      pltpu.sync_copy(dacc.at[pl.ds(r0, RPT)], deg_out.at[pl.ds(r0, RPT)])

  return pl.kernel(body, out_type=out_type, mesh=mesh,
                   scratch_types=tuple(scratch),
                   compiler_params=pltpu.CompilerParams(
                       use_tc_tiling_on_sc=False))


def _segsum_cls_kernel():
  mesh = plsc.VectorSubcoreMesh(core_axis_name="c", subcore_axis_name="s")
  out_type = jax.ShapeDtypeStruct((NC, NPAD, CPAD), jnp.float32)
  scratch = [
      pltpu.VMEM((CPT_B, K), jnp.int32),
      pltpu.VMEM((CPT_B, K), jnp.int32),
      pltpu.VMEM((NB_B, K, CPAD), jnp.float32),
      pltpu.VMEM((ZB, CPAD), jnp.float32),
      pltpu.VMEM_SHARED((NPAD, CPAD), jnp.float32),
  ] + [pltpu.SemaphoreType.DMA] * (2 * NB_B)

  def body(feat, src, dst, out, sidx, didx, rows, zbuf, acc, *sems):
    gsem = sems[:NB_B]
    ssem = sems[NB_B:2 * NB_B]
    c = lax.axis_index("c")
    s = lax.axis_index("s")
    w = c * NS + s

    _fill(zbuf, ZB, CPAD, 0.0)

    pltpu.sync_copy(src.at[pl.ds(w * CPT_B, CPT_B)], sidx)
    pltpu.sync_copy(dst.at[pl.ds(w * CPT_B, CPT_B)], didx)

    r0 = s * RPT

    def zero_loop(i, _):
      pltpu.sync_copy(zbuf, acc.at[pl.ds(r0 + i * ZB, ZB)])
      return 0

    lax.fori_loop(0, RPT // ZB, zero_loop, 0)
    plsc.subcore_barrier()

    for b in range(PF_B):
      pltpu.async_copy(feat.at[sidx.at[b]], rows.at[b], gsem[b])

    def outer(g, _):
      for b in range(NB_B):
        cs = g * NB_B + b
        bg = (b + PF_B) % NB_B

        @pl.when(jnp.logical_and(cs >= NB_B - PF_B, cs < CPT_B - PF_B))
        def _():
          pltpu.make_async_copy(rows.at[bg], acc.at[didx.at[0]],
                                ssem[bg]).wait()

        @pl.when(cs < CPT_B - PF_B)
        def _():
          pltpu.async_copy(feat.at[sidx.at[cs + PF_B]], rows.at[bg],
                           gsem[bg])

        pltpu.make_async_copy(feat.at[sidx.at[cs]], rows.at[b],
                              gsem[b]).wait()
        pltpu.async_copy(rows.at[b], acc.at[didx.at[cs]], ssem[b], add=True)
      return 0

    lax.fori_loop(0, CPT_B // NB_B, outer, 0)

    for b in range(NB_B):
      pltpu.make_async_copy(rows.at[b], acc.at[didx.at[0]], ssem[b]).wait()
    plsc.subcore_barrier()

    pltpu.sync_copy(acc.at[pl.ds(r0, RPT)], out.at[c, pl.ds(r0, RPT)])

  return pl.kernel(body, out_type=out_type, mesh=mesh,
                   scratch_types=tuple(scratch),
                   compiler_params=pltpu.CompilerParams(
                       use_tc_tiling_on_sc=False))


_segsum_feat = _segsum_feat_kernel()
_segsum_cls = _segsum_cls_kernel()

BR = 5120


def _bdot(a, b):
  return jax.lax.dot(a.astype(jnp.bfloat16), b.astype(jnp.bfloat16),
                     preferred_element_type=jnp.float32)


def _dense0_body(s0, degp, x, wl0, bl0, wr0, scale, shift, wl1, h_out, q_out):
  deg = degp[:, 0:1]
  mean = s0[...] / jnp.maximum(deg, 1.0)
  z = _bdot(mean, wl0[...]) + bl0[...] + _bdot(x[...], wr0[...])
  h = jnp.maximum(z * scale[...] + shift[...], 0.0)
  h_out[...] = h
  q_out[...] = _bdot(h, wl1[...])


def _dense0(s0, degp, x, wl0, bl0, wr0, scale, shift, wl1):
  grid = (NPAD // BR,)
  return pl.pallas_call(
      _dense0_body,
      grid=grid,
      in_specs=[
          pl.BlockSpec((BR, NFEAT), lambda i: (i, 0)),
          pl.BlockSpec((BR, 16), lambda i: (i, 0)),
          pl.BlockSpec((BR, NFEAT), lambda i: (i, 0)),
          pl.BlockSpec((NFEAT, NHID), lambda i: (0, 0)),
          pl.BlockSpec((1, NHID), lambda i: (0, 0)),
          pl.BlockSpec((NFEAT, NHID), lambda i: (0, 0)),
          pl.BlockSpec((1, NHID), lambda i: (0, 0)),
          pl.BlockSpec((1, NHID), lambda i: (0, 0)),
          pl.BlockSpec((NHID, CPAD), lambda i: (0, 0)),
      ],
      out_specs=[
          pl.BlockSpec((BR, NHID), lambda i: (i, 0)),
          pl.BlockSpec((BR, CPAD), lambda i: (i, 0)),
      ],
      out_shape=[
          jax.ShapeDtypeStruct((NPAD, NHID), jnp.float32),
          jax.ShapeDtypeStruct((NPAD, CPAD), jnp.float32),
      ],
  )(s0, degp, x, wl0, bl0, wr0, scale, shift, wl1)


def _dense1_body(s1p, degp, h, wr1, bl1, out):
  ssum = s1p[0] + s1p[1]
  deg = degp[:, 0:1]
  z = ssum / jnp.maximum(deg, 1.0) + bl1[...] + _bdot(h[...], wr1[...])
  mask = lax.broadcasted_iota(jnp.int32, (1, CPAD), 1) < NCLASS
  z = jnp.where(mask, z, -1e30)
  m = jnp.max(z, axis=1, keepdims=True)
  ez = jnp.exp(z - m)
  lse = jnp.log(jnp.sum(ez, axis=1, keepdims=True))
  out[...] = (z - m - lse)[:, :NCLASS]


def _dense1(s1p, degp, h, wr1, bl1):
  grid = (NPAD // BR,)
  return pl.pallas_call(
      _dense1_body,
      grid=grid,
      in_specs=[
          pl.BlockSpec((NC, BR, CPAD), lambda i: (0, i, 0)),
          pl.BlockSpec((BR, 16), lambda i: (i, 0)),
          pl.BlockSpec((BR, NHID), lambda i: (i, 0)),
          pl.BlockSpec((NHID, CPAD), lambda i: (0, 0)),
          pl.BlockSpec((1, CPAD), lambda i: (0, 0)),
      ],
      out_specs=pl.BlockSpec((BR, NCLASS), lambda i: (i, 0)),
      out_shape=jax.ShapeDtypeStruct((N, NCLASS), jnp.float32),
  )(s1p, degp, h, wr1, bl1)


def kernel(x, edge_index, W_l0, b_l0, W_r0, gamma0, beta0, W_l1, b_l1, W_r1):
  src = edge_index[0].reshape(E // K, K)
  dst = edge_index[1].reshape(E // K, K)
  x2 = x.reshape(2 * N, 64)
  s0, degp = _segsum_feat(x2, src, dst)

  scale = (gamma0 / jnp.sqrt(1.0 + BN_EPS)).reshape(1, NHID)
  shift = beta0.reshape(1, NHID)
  wl1 = jnp.pad(W_l1, ((0, 0), (0, CPAD - NCLASS)))
  xpad = jnp.pad(x, ((0, NPAD - N), (0, 0)))
  h, q = _dense0(s0, degp, xpad, W_l0, b_l0.reshape(1, NHID), W_r0,
                 scale, shift, wl1)

  s1p = _segsum_cls(q, src, dst)

  wr1 = jnp.pad(W_r1, ((0, 0), (0, CPAD - NCLASS)))
  bl1 = jnp.pad(b_l1, (0, CPAD - NCLASS)).reshape(1, CPAD)
  return _dense1(s1p, degp, h, wr1, bl1)
